# baseline (speedup 1.0000x reference)
import functools

import jax
import jax.numpy as jnp
from jax import lax
from jax.experimental import pallas as pl
from jax.experimental.pallas import tpu as pltpu
from jax.experimental.pallas import tpu_sc as plsc

N = 10000
E = 160000
E_PAD = 163840
N_T = N + 1
NC, NS = 2, 16
N_ACC = 10112
RPT = N_ACC // NS
B = 128
GATW = 144
GATB = 160


def _gat_sc(table, sdst, src2d, dst2d, *, lpg, loff, cph):
    ept = E_PAD // NS
    nbatch = ept // B
    hb = nbatch // 2
    BF = jnp.bfloat16
    mesh = plsc.VectorSubcoreMesh(core_axis_name="c", subcore_axis_name="s")

    @functools.partial(
        pl.kernel,
        out_type=jax.ShapeDtypeStruct((NC * N_ACC, GATB), BF),
        mesh=mesh,
        scratch_types=[
            pltpu.VMEM((hb, B), jnp.int32),
            pltpu.VMEM((hb, B), jnp.int32),
            pltpu.VMEM((B, GATW), jnp.float32),
            pltpu.VMEM((B, GATW), jnp.float32),
            pltpu.VMEM((B, GATB), BF),
            pltpu.VMEM((B, GATB), BF),
            pltpu.VMEM((B, 16), jnp.float32),
            pltpu.VMEM((B, 16), jnp.float32),
            pltpu.VMEM_SHARED((N_ACC, GATB), BF),
        ] + [pltpu.SemaphoreType.DMA] * 6,
        compiler_params=pltpu.CompilerParams(use_tc_tiling_on_sc=False,
                                             needs_layout_passes=False),
    )
    def k(table_hbm, sdst_hbm, src_hbm, dst_hbm, out_hbm,
          srcb, dstb, rows0, rows1, st0, st1, sd0, sd1, acc,
          gs0, gs1, ds0, ds1, cs0, cs1):
        g = lax.axis_index("c")
        s = lax.axis_index("s")
        zeros32 = jnp.zeros((32,), BF)
        rows_ = (rows0, rows1)
        st_ = (st0, st1)
        sd_ = (sd0, sd1)
        gs_ = (gs0, gs1)
        ds_ = (ds0, ds1)
        cs_ = (cs0, cs1)
        rowoff = g * N_T

        def zrow(i, c):
            for kk in range(GATB // 32):
                st0[i, pl.ds(32 * kk, 32)] = zeros32
            return c
        lax.fori_loop(0, B, zrow, 0)
        base0 = s * RPT
        full, rem = RPT // B, RPT % B
        for j in range(full):
            pltpu.sync_copy(st0, acc.at[pl.ds(base0 + j * B, B)])
        if rem:
            pltpu.sync_copy(st0.at[pl.ds(0, rem)],
                            acc.at[pl.ds(base0 + full * B, rem)])
        plsc.subcore_barrier()

        def g_rows(sl, b):
            return pltpu.make_async_copy(table_hbm.at[srcb.at[b]],
                                         rows_[sl], gs_[sl])

        def g_sd(sl, b):
            return pltpu.make_async_copy(sdst_hbm.at[dstb.at[b]],
                                         sd_[sl], ds_[sl])

        def g_sc(sl, b):
            return pltpu.make_async_copy(st_[sl], acc.at[dstb.at[b]],
                                         cs_[sl])

        def compute(sl):
            rows = rows_[sl]
            st = st_[sl]
            sd = sd_[sl]

            @plsc.parallel_loop(0, B, 1, unroll=4)
            def edge(e):
                sv = rows[e, pl.ds(128, 16)]
                dv = sd[e, :]
                z = sv + dv
                w = jnp.exp(jnp.maximum(z, 0.2 * z))
                st[e, pl.ds(128, 32)] = plsc.pack(
                    w, w, format=plsc.PackFormat.INTERLEAVED)
                for hh in range(128 // (16 * cph)):
                    lane = g * lpg + (loff + hh)
                    wk = w.at[jnp.full((16,), lane, jnp.int32)].get(
                        mode="promise_in_bounds")
                    for p in range(cph // 2):
                        k0 = hh * cph + 2 * p
                        va = rows[e, pl.ds(16 * k0, 16)] * wk
                        vb = rows[e, pl.ds(16 * k0 + 16, 16)] * wk
                        st[e, pl.ds(16 * k0, 32)] = plsc.pack(
                            va, vb, format=plsc.PackFormat.INTERLEAVED)

        def step(b, sl):
            other = 1 - sl
            nb = b + 1

            @pl.when(nb < hb)
            def _issue():
                @pl.when(nb >= 2)
                def _drain():
                    g_sc(other, 0).wait()
                g_rows(other, nb).start()
                g_sd(other, nb).start()

            g_rows(sl, b).wait()
            g_sd(sl, b).wait()
            compute(sl)
            g_sc(sl, b).start(add=True)

        def body(i, c):
            step(2 * i, 0)
            step(2 * i + 1, 1)
            return c

        for half in range(2):
            rbase = s * nbatch + half * hb
            pltpu.sync_copy(src_hbm.at[pl.ds(rbase, hb)], srcb)
            pltpu.sync_copy(dst_hbm.at[pl.ds(rbase, hb)], dstb)

            def addoff(r, c):
                for j in range(B // 16):
                    srcb[r, pl.ds(16 * j, 16)] = (
                        srcb[r, pl.ds(16 * j, 16)] + rowoff)
                return c
            lax.fori_loop(0, hb, addoff, 0)

            g_rows(0, 0).start()
            g_sd(0, 0).start()
            lax.fori_loop(0, hb // 2, body, 0)
            g_sc(0, 0).wait()
            g_sc(1, 0).wait()

        plsc.subcore_barrier()
        pltpu.sync_copy(acc.at[pl.ds(base0, RPT)],
                        out_hbm.at[pl.ds(g * N_ACC + base0, RPT)])

    return k(table, sdst, src2d, dst2d)


def _gcn_sc(table, src3d, dst3d, width):
    BG = 128
    ept = E_PAD // (NC * NS)
    nbatch = ept // BG
    nslot = 2
    mesh = plsc.VectorSubcoreMesh(core_axis_name="c", subcore_axis_name="s")

    @functools.partial(
        pl.kernel,
        out_type=jax.ShapeDtypeStruct((NC * N_ACC, width), jnp.float32),
        mesh=mesh,
        scratch_types=[
            pltpu.VMEM((nbatch, BG), jnp.int32),
            pltpu.VMEM((nbatch, BG), jnp.int32),
        ] + [pltpu.VMEM((BG, width), jnp.float32)] * nslot
          + [pltpu.VMEM_SHARED((N_ACC, width), jnp.float32)]
          + [pltpu.SemaphoreType.DMA] * (2 * nslot),
        compiler_params=pltpu.CompilerParams(use_tc_tiling_on_sc=False),
    )
    def k(table_hbm, src_hbm, dst_hbm, out_hbm, srcb, dstb, *scr):
        rows_ = scr[:nslot]
        acc = scr[nslot]
        gs_ = scr[nslot + 1:2 * nslot + 1]
        cs_ = scr[2 * nslot + 1:]
        g = lax.axis_index("c")
        s = lax.axis_index("s")
        wid = g * NS + s
        zeros16 = jnp.zeros((16,), jnp.float32)

        rb = wid * nbatch
        pltpu.sync_copy(src_hbm.at[pl.ds(rb, nbatch)], srcb)
        pltpu.sync_copy(dst_hbm.at[pl.ds(rb, nbatch)], dstb)

        zbuf = rows_[0]

        def zrow(i, c):
            for kk in range(width // 16):
                zbuf[i, pl.ds(16 * kk, 16)] = zeros16
            return c
        lax.fori_loop(0, BG, zrow, 0)
        base0 = s * RPT
        full, rem = RPT // BG, RPT % BG
        for j in range(full):
            pltpu.sync_copy(zbuf, acc.at[pl.ds(base0 + j * BG, BG)])
        if rem:
            pltpu.sync_copy(zbuf.at[pl.ds(0, rem)],
                            acc.at[pl.ds(base0 + full * BG, rem)])
        plsc.subcore_barrier()

        def g_rows(sl, b):
            return pltpu.make_async_copy(table_hbm.at[srcb.at[b]],
                                         rows_[sl], gs_[sl])

        def g_sc(sl, b):
            return pltpu.make_async_copy(rows_[sl], acc.at[dstb.at[b]],
                                         cs_[sl])

        def step(t, sl):
            other = 1 - sl
            nb = t + 1

            @pl.when(nb < nbatch)
            def _issue():
                @pl.when(nb >= 2)
                def _drain():
                    g_sc(other, 0).wait()
                g_rows(other, nb).start()

            g_rows(sl, t).wait()
            g_sc(sl, t).start(add=True)

        g_rows(0, 0).start()

        def body(i, c):
            step(2 * i, 0)
            step(2 * i + 1, 1)
            return c
        lax.fori_loop(0, nbatch // 2, body, 0)
        g_sc(0, 0).wait()
        g_sc(1, 0).wait()

        plsc.subcore_barrier()
        pltpu.sync_copy(acc.at[pl.ds(base0, RPT)],
                        out_hbm.at[pl.ds(g * N_ACC + base0, RPT)])

    return k(table, src3d, dst3d)


RB = 1000
F32 = jnp.float32


def _node_spec(width):
    return pl.BlockSpec((RB, width), lambda i: (i, 0))


def _full_spec(r, c):
    return pl.BlockSpec((r, c), lambda i: (0, 0))


def _out(width):
    return jax.ShapeDtypeStruct((N, width), F32)


def _tc1(x, wcat, acat):
    def body(x_ref, w_ref, a_ref, t0_ref, t1_ref, sd_ref, mm_ref):
        xb = x_ref[...]
        hcat = jnp.dot(xb, w_ref[...], preferred_element_type=F32)
        h1 = hcat[:, :256]
        mm_ref[...] = hcat[:, 256:]
        sall = jnp.dot(h1, a_ref[...], preferred_element_type=F32)
        zp = jnp.zeros((RB, 8), F32)
        t0_ref[...] = jnp.concatenate([h1[:, :128], sall[:, :8], zp], axis=1)
        t1_ref[...] = jnp.concatenate([h1[:, 128:], sall[:, :8], zp], axis=1)
        sd_ref[...] = jnp.concatenate([sall[:, 8:], zp], axis=1)

    return pl.pallas_call(
        body,
        grid=(N // RB,),
        in_specs=[_node_spec(1024), _full_spec(1024, 512), _full_spec(256, 16)],
        out_specs=[_node_spec(GATW), _node_spec(GATW), _node_spec(16),
                   _node_spec(256)],
        out_shape=[_out(GATW), _out(GATW), _out(16), _out(256)],
    )(x, wcat, acat)


def _tc2(a0, a1, t0, t1, sd1, mm1, pv, w2cat):
    def body(a0_ref, a1_ref, t0_ref, t1_ref, sd_ref, mm_ref, pv_ref, w_ref,
             t2_ref, h2_ref, mm2_ref, dv_ref):
        a0b, a1b = a0_ref[...], a1_ref[...]
        t0b, t1b = t0_ref[...], t1_ref[...]
        rep = (lax.broadcasted_iota(jnp.int32, (4, 128), 1) // 32 ==
               lax.broadcasted_iota(jnp.int32, (4, 128), 0)).astype(F32)
        ssrc = t0b[:, 128:136]
        sdst = sd_ref[...][:, :8]
        z = ssrc + sdst
        wself = jnp.exp(jnp.maximum(z, 0.2 * z))
        deg = a0b[:, 136:137] + 1.0
        dinv = lax.rsqrt(deg)
        num0 = a0b[:, :128] + t0b[:, :128] * jnp.dot(wself[:, :4], rep)
        den0 = jnp.dot(a0b[:, 128:132] + wself[:, :4], rep)
        num1 = a1b[:, :128] + t1b[:, :128] * jnp.dot(wself[:, 4:], rep)
        den1 = jnp.dot(a1b[:, 132:136] + wself[:, 4:], rep)
        gat = jnp.concatenate([num0 / den0, num1 / den1], axis=1)
        pvb = pv_ref[...]
        o = gat + mm_ref[...] + pvb[0] + pvb[1]
        mu = o.mean(-1, keepdims=True)
        var = ((o - mu) ** 2).mean(-1, keepdims=True)
        h = jnp.maximum((o - mu) * lax.rsqrt(var + 1e-5) * pvb[2] + pvb[3], 0.0)
        hcat = jnp.dot(h, w_ref[...], preferred_element_type=F32)
        h2 = hcat[:, :128]
        h2_ref[...] = h2
        mm2_ref[...] = hcat[:, 128:]
        t2_ref[...] = h2 * dinv
        dv_ref[...] = jnp.broadcast_to(dinv, (RB, 8))

    return pl.pallas_call(
        body,
        grid=(N // RB,),
        in_specs=[_node_spec(GATW), _node_spec(GATW), _node_spec(GATW),
                  _node_spec(GATW), _node_spec(16), _node_spec(256),
                  _full_spec(4, 256), _full_spec(256, 256)],
        out_specs=[_node_spec(128), _node_spec(128), _node_spec(128),
                   _node_spec(8)],
        out_shape=[_out(128), _out(128), _out(128), _out(8)],
    )(a0, a1, t0, t1, sd1, mm1, pv, w2cat)


def _tc3(p0, p1, h2, mm2, dv8, pv, w3, acat3, wm3):
    def body(p0_ref, p1_ref, h2_ref, mm2_ref, dv_ref, pv_ref, w3_ref, a_ref,
             wm_ref, t30_ref, t31_ref, t32_ref, t33_ref, sd_ref, mm3_ref):
        dinv = dv_ref[...][:, :1]
        gcn = (p0_ref[...] + p1_ref[...]) * dinv + h2_ref[...] * dinv * dinv
        pvb = pv_ref[...]
        o = gcn + mm2_ref[...] + pvb[0] + pvb[1]
        mu = o.mean(-1, keepdims=True)
        var = ((o - mu) ** 2).mean(-1, keepdims=True)
        h = jnp.maximum((o - mu) * lax.rsqrt(var + 1e-5) * pvb[2] + pvb[3], 0.0)
        h3 = jnp.dot(h, w3_ref[...], preferred_element_type=F32)
        sall = jnp.dot(h3, a_ref[...], preferred_element_type=F32)
        mm3_ref[...] = jnp.dot(h, wm_ref[...], preferred_element_type=F32)
        zp = jnp.zeros((RB, 8), F32)
        for g, tref in enumerate((t30_ref, t31_ref, t32_ref, t33_ref)):
            tref[...] = jnp.concatenate(
                [h3[:, 128 * g:128 * (g + 1)], sall[:, :8], zp], axis=1)
        sd_ref[...] = jnp.concatenate([sall[:, 8:], zp], axis=1)

    return pl.pallas_call(
        body,
        grid=(N // RB,),
        in_specs=[_node_spec(128), _node_spec(128), _node_spec(128),
                  _node_spec(128), _node_spec(8), _full_spec(4, 128),
                  _full_spec(128, 512), _full_spec(512, 16),
                  _full_spec(128, 64)],
        out_specs=[_node_spec(GATW)] * 4 + [_node_spec(16), _node_spec(64)],
        out_shape=[_out(GATW)] * 4 + [_out(16), _out(64)],
    )(p0, p1, h2, mm2, dv8, pv, w3, acat3, wm3)


def _tc4(accs, t3s, sd3, mm3, dv8, pv):
    def body(a0_ref, a1_ref, a2_ref, a3_ref, t0_ref, t1_ref, t2_ref, t3_ref,
             sd_ref, mm_ref, dv_ref, pv_ref, t4_ref, h4_ref):
        arefs = (a0_ref, a1_ref, a2_ref, a3_ref)
        trefs = (t0_ref, t1_ref, t2_ref, t3_ref)
        rep = (lax.broadcasted_iota(jnp.int32, (2, 128), 1) // 64 ==
               lax.broadcasted_iota(jnp.int32, (2, 128), 0)).astype(F32)
        mean8 = (lax.broadcasted_iota(jnp.int32, (512, 64), 0) % 64 ==
                 lax.broadcasted_iota(jnp.int32, (512, 64), 1)).astype(F32) / 8.0
        ssrc = t0_ref[...][:, 128:136]
        z = ssrc + sd_ref[...][:, :8]
        wself = jnp.exp(jnp.maximum(z, 0.2 * z))
        ratios = []
        for g in range(4):
            ab, tb = arefs[g][...], trefs[g][...]
            ws2 = wself[:, 2 * g:2 * g + 2]
            num = ab[:, :128] + tb[:, :128] * jnp.dot(ws2, rep)
            den = jnp.dot(ab[:, 128 + 2 * g:130 + 2 * g] + ws2, rep)
            ratios.append(num / den)
        rat = jnp.concatenate(ratios, axis=1)
        out64 = jnp.dot(rat, mean8, preferred_element_type=F32)
        pvb = pv_ref[...]
        o = out64 + mm_ref[...] + pvb[0] + pvb[1]
        mu = o.mean(-1, keepdims=True)
        var = ((o - mu) ** 2).mean(-1, keepdims=True)
        h = jnp.maximum((o - mu) * lax.rsqrt(var + 1e-5) * pvb[2] + pvb[3], 0.0)
        t4_ref[...] = h * dv_ref[...][:, :1]
        h4_ref[...] = h

    return pl.pallas_call(
        body,
        grid=(N // RB,),
        in_specs=[_node_spec(GATW)] * 8 + [_node_spec(16), _node_spec(64),
                                           _node_spec(8), _full_spec(4, 64)],
        out_specs=[_node_spec(64), _node_spec(64)],
        out_shape=[_out(64), _out(64)],
    )(*accs, *t3s, sd3, mm3, dv8, pv)


def _tc5(p0, p1, h4, dv8, w4, wm4, bsum):
    def body(p0_ref, p1_ref, h4_ref, dv_ref, w4_ref, wm_ref, b_ref, o_ref):
        dinv = dv_ref[...][:, :1]
        h4b = h4_ref[...]
        gcn = (p0_ref[...] + p1_ref[...]) * dinv + h4b * dinv * dinv
        o_ref[...] = (jnp.dot(gcn, w4_ref[...], preferred_element_type=F32) +
                      jnp.dot(h4b, wm_ref[...], preferred_element_type=F32) +
                      b_ref[...])

    return pl.pallas_call(
        body,
        grid=(N // RB,),
        in_specs=[_node_spec(64), _node_spec(64), _node_spec(64),
                  _node_spec(8), _full_spec(64, 2), _full_spec(64, 2),
                  _full_spec(1, 2)],
        out_specs=_node_spec(2),
        out_shape=_out(2),
    )(p0, p1, h4, dv8, w4, wm4, bsum)


def _acat(a_src, a_dst, out_ch):
    c = 8 * out_ch
    hot = (jnp.arange(c)[:, None] // out_ch == jnp.arange(8)[None, :]
           ).astype(F32)
    return jnp.concatenate([a_src.reshape(-1)[:, None] * hot,
                            a_dst.reshape(-1)[:, None] * hot], axis=1)


def kernel(x, edge_index, W1, a_src1, a_dst1, b1, Wm1, bm1, g0, be0,
           W2, b2, Wm2, bm2, g1, be1, W3, a_src3, a_dst3, b3, Wm3, bm3,
           g2, be2, W4, b4, Wm4, bm4):
    npad = E_PAD - E
    src32 = edge_index[0].astype(jnp.int32)
    dst32 = edge_index[1].astype(jnp.int32)
    src = jnp.concatenate([src32,
                           jnp.full((npad,), N, jnp.int32)]).reshape(-1, B)
    dst = jnp.concatenate([dst32,
                           jnp.zeros((npad,), jnp.int32)]).reshape(-1, B)
    srcg = jnp.concatenate([src32,
                            jnp.zeros((npad,), jnp.int32)]).reshape(-1, 128)
    dstg = jnp.concatenate([dst32,
                            jnp.full((npad,), N, jnp.int32)]).reshape(-1, 128)

    dummy_gat = jnp.concatenate([jnp.zeros((128,), F32),
                                 jnp.full((16,), -1e30, F32)])[None]

    def unswz(a):
        m = a.shape[0]
        au = (a.astype(F32).reshape(m, GATB // 32, 16, 2)
              .swapaxes(2, 3).reshape(m, GATB))
        return au[:, :GATW]

    t0, t1, sd1, mm1 = _tc1(x, jnp.concatenate([W1, Wm1], axis=1),
                            _acat(a_src1, a_dst1, 32))
    table1 = jnp.concatenate([t0, dummy_gat, t1, dummy_gat], axis=0)
    acc1 = unswz(_gat_sc(table1, sd1, src, dst, lpg=4, loff=0, cph=2))
    t2, h2, mm2, dv8 = _tc2(acc1[:N], acc1[N_ACC:N_ACC + N], t0, t1, sd1, mm1,
                            jnp.stack([b1, bm1, g0, be0]),
                            jnp.concatenate([W2, Wm2], axis=1))

    p2 = _gcn_sc(t2, srcg, dstg, 128)
    t30, t31, t32, t33, sd3, mm3 = _tc3(p2[:N], p2[N_ACC:N_ACC + N], h2,
                                        mm2, dv8,
                                        jnp.stack([b2, bm2, g1, be1]),
                                        W3, _acat(a_src3, a_dst3, 64), Wm3)

    tableA = jnp.concatenate([t30, dummy_gat, t31, dummy_gat], axis=0)
    tableB = jnp.concatenate([t32, dummy_gat, t33, dummy_gat], axis=0)
    accA = unswz(_gat_sc(tableA, sd3, src, dst, lpg=2, loff=0, cph=4))
    accB = unswz(_gat_sc(tableB, sd3, src, dst, lpg=2, loff=4, cph=4))
    t4, h4 = _tc4((accA[:N], accA[N_ACC:N_ACC + N],
                   accB[:N], accB[N_ACC:N_ACC + N]),
                  (t30, t31, t32, t33), sd3, mm3, dv8,
                  jnp.stack([b3, bm3, g2, be2]))

    p4 = _gcn_sc(t4, srcg, dstg, 64)
    return _tc5(p4[:N], p4[N_ACC:N_ACC + N], h4, dv8, W4, Wm4,
                (b4 + bm4)[None])


# speedup vs baseline: 1.0648x; 1.0648x over previous
import functools

import jax
import jax.numpy as jnp
from jax import lax
from jax.experimental import pallas as pl
from jax.experimental.pallas import tpu as pltpu
from jax.experimental.pallas import tpu_sc as plsc

N = 10000
E = 160000
E_PAD = 163840
N_T = N + 1
NC, NS = 2, 16
N_ACC = 10112
RPT = N_ACC // NS
B = 64
GATW = 144


def _gat_sc(table, sdst, src2d, dst2d, *, lpg, loff, cph):
    ept = E_PAD // NS
    nbatch = ept // B
    hb = nbatch // 2
    mesh = plsc.VectorSubcoreMesh(core_axis_name="c", subcore_axis_name="s")

    @functools.partial(
        pl.kernel,
        out_type=jax.ShapeDtypeStruct((NC * N_ACC, GATW), jnp.float32),
        mesh=mesh,
        scratch_types=[
            pltpu.VMEM((hb, B), jnp.int32),
            pltpu.VMEM((hb, B), jnp.int32),
            pltpu.VMEM((B, GATW), jnp.float32),
            pltpu.VMEM((B, GATW), jnp.float32),
            pltpu.VMEM((B, 16), jnp.float32),
            pltpu.VMEM((B, 16), jnp.float32),
            pltpu.VMEM_SHARED((N_ACC, GATW), jnp.float32),
        ] + [pltpu.SemaphoreType.DMA] * 6,
        compiler_params=pltpu.CompilerParams(use_tc_tiling_on_sc=False),
    )
    def k(table_hbm, sdst_hbm, src_hbm, dst_hbm, out_hbm,
          srcb, dstb, rows0, rows1, sd0, sd1, acc,
          gs0, gs1, ds0, ds1, cs0, cs1):
        g = lax.axis_index("c")
        s = lax.axis_index("s")
        zeros16 = jnp.zeros((16,), jnp.float32)
        rows_ = (rows0, rows1)
        sd_ = (sd0, sd1)
        gs_ = (gs0, gs1)
        ds_ = (ds0, ds1)
        cs_ = (cs0, cs1)
        rowoff = g * N_T

        def zrow(i, c):
            for kk in range(GATW // 16):
                rows0[i, pl.ds(16 * kk, 16)] = zeros16
            return c
        lax.fori_loop(0, B, zrow, 0)
        base0 = s * RPT
        full, rem = RPT // B, RPT % B
        for j in range(full):
            pltpu.sync_copy(rows0, acc.at[pl.ds(base0 + j * B, B)])
        if rem:
            pltpu.sync_copy(rows0.at[pl.ds(0, rem)],
                            acc.at[pl.ds(base0 + full * B, rem)])
        plsc.subcore_barrier()

        def g_rows(sl, b):
            return pltpu.make_async_copy(table_hbm.at[srcb.at[b]],
                                         rows_[sl], gs_[sl])

        def g_sd(sl, b):
            return pltpu.make_async_copy(sdst_hbm.at[dstb.at[b]],
                                         sd_[sl], ds_[sl])

        def g_sc(sl, b):
            return pltpu.make_async_copy(rows_[sl], acc.at[dstb.at[b]],
                                         cs_[sl])

        def compute(sl):
            rows = rows_[sl]
            sd = sd_[sl]

            @plsc.parallel_loop(0, B, 1, unroll=8)
            def edge(e):
                sv = rows[e, pl.ds(128, 16)]
                dv = sd[e, :]
                z = sv + dv
                w = jnp.exp(jnp.maximum(z, 0.2 * z))
                rows[e, pl.ds(128, 16)] = w
                for hh in range(128 // (16 * cph)):
                    lane = g * lpg + (loff + hh)
                    wk = w.at[jnp.full((16,), lane, jnp.int32)].get(
                        mode="promise_in_bounds")
                    for p in range(cph):
                        kk = hh * cph + p
                        rows[e, pl.ds(16 * kk, 16)] = (
                            rows[e, pl.ds(16 * kk, 16)] * wk)

        def step(b, sl):
            other = 1 - sl
            nb = b + 1

            @pl.when(nb < hb)
            def _issue():
                @pl.when(nb >= 2)
                def _drain():
                    g_sc(other, 0).wait()
                g_rows(other, nb).start()
                g_sd(other, nb).start()

            g_rows(sl, b).wait()
            g_sd(sl, b).wait()
            compute(sl)
            g_sc(sl, b).start(add=True)

        def body(i, c):
            step(2 * i, 0)
            step(2 * i + 1, 1)
            return c

        for half in range(2):
            rbase = s * nbatch + half * hb
            pltpu.sync_copy(src_hbm.at[pl.ds(rbase, hb)], srcb)
            pltpu.sync_copy(dst_hbm.at[pl.ds(rbase, hb)], dstb)

            def addoff(r, c):
                for j in range(B // 16):
                    srcb[r, pl.ds(16 * j, 16)] = (
                        srcb[r, pl.ds(16 * j, 16)] + rowoff)
                return c
            lax.fori_loop(0, hb, addoff, 0)

            g_rows(0, 0).start()
            g_sd(0, 0).start()
            lax.fori_loop(0, hb // 2, body, 0)
            g_sc(0, 0).wait()
            g_sc(1, 0).wait()

        plsc.subcore_barrier()
        pltpu.sync_copy(acc.at[pl.ds(base0, RPT)],
                        out_hbm.at[pl.ds(g * N_ACC + base0, RPT)])

    return k(table, sdst, src2d, dst2d)


def _gcn_sc(table, src3d, dst3d, width):
    BG = 128
    ept = E_PAD // (NC * NS)
    nbatch = ept // BG
    nslot = 2
    mesh = plsc.VectorSubcoreMesh(core_axis_name="c", subcore_axis_name="s")

    @functools.partial(
        pl.kernel,
        out_type=jax.ShapeDtypeStruct((NC * N_ACC, width), jnp.float32),
        mesh=mesh,
        scratch_types=[
            pltpu.VMEM((nbatch, BG), jnp.int32),
            pltpu.VMEM((nbatch, BG), jnp.int32),
        ] + [pltpu.VMEM((BG, width), jnp.float32)] * nslot
          + [pltpu.VMEM_SHARED((N_ACC, width), jnp.float32)]
          + [pltpu.SemaphoreType.DMA] * (2 * nslot),
        compiler_params=pltpu.CompilerParams(use_tc_tiling_on_sc=False),
    )
    def k(table_hbm, src_hbm, dst_hbm, out_hbm, srcb, dstb, *scr):
        rows_ = scr[:nslot]
        acc = scr[nslot]
        gs_ = scr[nslot + 1:2 * nslot + 1]
        cs_ = scr[2 * nslot + 1:]
        g = lax.axis_index("c")
        s = lax.axis_index("s")
        wid = g * NS + s
        zeros16 = jnp.zeros((16,), jnp.float32)

        rb = wid * nbatch
        pltpu.sync_copy(src_hbm.at[pl.ds(rb, nbatch)], srcb)
        pltpu.sync_copy(dst_hbm.at[pl.ds(rb, nbatch)], dstb)

        zbuf = rows_[0]

        def zrow(i, c):
            for kk in range(width // 16):
                zbuf[i, pl.ds(16 * kk, 16)] = zeros16
            return c
        lax.fori_loop(0, BG, zrow, 0)
        base0 = s * RPT
        full, rem = RPT // BG, RPT % BG
        for j in range(full):
            pltpu.sync_copy(zbuf, acc.at[pl.ds(base0 + j * BG, BG)])
        if rem:
            pltpu.sync_copy(zbuf.at[pl.ds(0, rem)],
                            acc.at[pl.ds(base0 + full * BG, rem)])
        plsc.subcore_barrier()

        def g_rows(sl, b):
            return pltpu.make_async_copy(table_hbm.at[srcb.at[b]],
                                         rows_[sl], gs_[sl])

        def g_sc(sl, b):
            return pltpu.make_async_copy(rows_[sl], acc.at[dstb.at[b]],
                                         cs_[sl])

        def step(t, sl):
            other = 1 - sl
            nb = t + 1

            @pl.when(nb < nbatch)
            def _issue():
                @pl.when(nb >= 2)
                def _drain():
                    g_sc(other, 0).wait()
                g_rows(other, nb).start()

            g_rows(sl, t).wait()
            g_sc(sl, t).start(add=True)

        g_rows(0, 0).start()

        def body(i, c):
            step(2 * i, 0)
            step(2 * i + 1, 1)
            return c
        lax.fori_loop(0, nbatch // 2, body, 0)
        g_sc(0, 0).wait()
        g_sc(1, 0).wait()

        plsc.subcore_barrier()
        pltpu.sync_copy(acc.at[pl.ds(base0, RPT)],
                        out_hbm.at[pl.ds(g * N_ACC + base0, RPT)])

    return k(table, src3d, dst3d)


RB = 1000
F32 = jnp.float32


def _node_spec(width):
    return pl.BlockSpec((RB, width), lambda i: (i, 0))


def _full_spec(r, c):
    return pl.BlockSpec((r, c), lambda i: (0, 0))


def _out(width):
    return jax.ShapeDtypeStruct((N, width), F32)


def _tc1(x, wcat, acat):
    def body(x_ref, w_ref, a_ref, t0_ref, t1_ref, sd_ref, mm_ref):
        xb = x_ref[...]
        hcat = jnp.dot(xb, w_ref[...], preferred_element_type=F32)
        h1 = hcat[:, :256]
        mm_ref[...] = hcat[:, 256:]
        sall = jnp.dot(h1, a_ref[...], preferred_element_type=F32)
        zp = jnp.zeros((RB, 8), F32)
        t0_ref[...] = jnp.concatenate([h1[:, :128], sall[:, :8], zp], axis=1)
        t1_ref[...] = jnp.concatenate([h1[:, 128:], sall[:, :8], zp], axis=1)
        sd_ref[...] = jnp.concatenate([sall[:, 8:], zp], axis=1)

    return pl.pallas_call(
        body,
        grid=(N // RB,),
        in_specs=[_node_spec(1024), _full_spec(1024, 512), _full_spec(256, 16)],
        out_specs=[_node_spec(GATW), _node_spec(GATW), _node_spec(16),
                   _node_spec(256)],
        out_shape=[_out(GATW), _out(GATW), _out(16), _out(256)],
    )(x, wcat, acat)


def _tc2(a0, a1, t0, t1, sd1, mm1, pv, w2cat):
    def body(a0_ref, a1_ref, t0_ref, t1_ref, sd_ref, mm_ref, pv_ref, w_ref,
             t2_ref, h2_ref, mm2_ref, dv_ref):
        a0b, a1b = a0_ref[...], a1_ref[...]
        t0b, t1b = t0_ref[...], t1_ref[...]
        rep = (lax.broadcasted_iota(jnp.int32, (4, 128), 1) // 32 ==
               lax.broadcasted_iota(jnp.int32, (4, 128), 0)).astype(F32)
        ssrc = t0b[:, 128:136]
        sdst = sd_ref[...][:, :8]
        z = ssrc + sdst
        wself = jnp.exp(jnp.maximum(z, 0.2 * z))
        deg = a0b[:, 136:137] + 1.0
        dinv = lax.rsqrt(deg)
        num0 = a0b[:, :128] + t0b[:, :128] * jnp.dot(wself[:, :4], rep)
        den0 = jnp.dot(a0b[:, 128:132] + wself[:, :4], rep)
        num1 = a1b[:, :128] + t1b[:, :128] * jnp.dot(wself[:, 4:], rep)
        den1 = jnp.dot(a1b[:, 132:136] + wself[:, 4:], rep)
        gat = jnp.concatenate([num0 / den0, num1 / den1], axis=1)
        pvb = pv_ref[...]
        o = gat + mm_ref[...] + pvb[0] + pvb[1]
        mu = o.mean(-1, keepdims=True)
        var = ((o - mu) ** 2).mean(-1, keepdims=True)
        h = jnp.maximum((o - mu) * lax.rsqrt(var + 1e-5) * pvb[2] + pvb[3], 0.0)
        hcat = jnp.dot(h, w_ref[...], preferred_element_type=F32)
        h2 = hcat[:, :128]
        h2_ref[...] = h2
        mm2_ref[...] = hcat[:, 128:]
        t2_ref[...] = h2 * dinv
        dv_ref[...] = jnp.broadcast_to(dinv, (RB, 8))

    return pl.pallas_call(
        body,
        grid=(N // RB,),
        in_specs=[_node_spec(GATW), _node_spec(GATW), _node_spec(GATW),
                  _node_spec(GATW), _node_spec(16), _node_spec(256),
                  _full_spec(4, 256), _full_spec(256, 256)],
        out_specs=[_node_spec(128), _node_spec(128), _node_spec(128),
                   _node_spec(8)],
        out_shape=[_out(128), _out(128), _out(128), _out(8)],
    )(a0, a1, t0, t1, sd1, mm1, pv, w2cat)


def _tc3(p0, p1, h2, mm2, dv8, pv, w3, acat3, wm3):
    def body(p0_ref, p1_ref, h2_ref, mm2_ref, dv_ref, pv_ref, w3_ref, a_ref,
             wm_ref, t30_ref, t31_ref, t32_ref, t33_ref, sd_ref, mm3_ref):
        dinv = dv_ref[...][:, :1]
        gcn = (p0_ref[...] + p1_ref[...]) * dinv + h2_ref[...] * dinv * dinv
        pvb = pv_ref[...]
        o = gcn + mm2_ref[...] + pvb[0] + pvb[1]
        mu = o.mean(-1, keepdims=True)
        var = ((o - mu) ** 2).mean(-1, keepdims=True)
        h = jnp.maximum((o - mu) * lax.rsqrt(var + 1e-5) * pvb[2] + pvb[3], 0.0)
        h3 = jnp.dot(h, w3_ref[...], preferred_element_type=F32)
        sall = jnp.dot(h3, a_ref[...], preferred_element_type=F32)
        mm3_ref[...] = jnp.dot(h, wm_ref[...], preferred_element_type=F32)
        zp = jnp.zeros((RB, 8), F32)
        for g, tref in enumerate((t30_ref, t31_ref, t32_ref, t33_ref)):
            tref[...] = jnp.concatenate(
                [h3[:, 128 * g:128 * (g + 1)], sall[:, :8], zp], axis=1)
        sd_ref[...] = jnp.concatenate([sall[:, 8:], zp], axis=1)

    return pl.pallas_call(
        body,
        grid=(N // RB,),
        in_specs=[_node_spec(128), _node_spec(128), _node_spec(128),
                  _node_spec(128), _node_spec(8), _full_spec(4, 128),
                  _full_spec(128, 512), _full_spec(512, 16),
                  _full_spec(128, 64)],
        out_specs=[_node_spec(GATW)] * 4 + [_node_spec(16), _node_spec(64)],
        out_shape=[_out(GATW)] * 4 + [_out(16), _out(64)],
    )(p0, p1, h2, mm2, dv8, pv, w3, acat3, wm3)


def _tc4(accs, t3s, sd3, mm3, dv8, pv):
    def body(a0_ref, a1_ref, a2_ref, a3_ref, t0_ref, t1_ref, t2_ref, t3_ref,
             sd_ref, mm_ref, dv_ref, pv_ref, t4_ref, h4_ref):
        arefs = (a0_ref, a1_ref, a2_ref, a3_ref)
        trefs = (t0_ref, t1_ref, t2_ref, t3_ref)
        rep = (lax.broadcasted_iota(jnp.int32, (2, 128), 1) // 64 ==
               lax.broadcasted_iota(jnp.int32, (2, 128), 0)).astype(F32)
        mean8 = (lax.broadcasted_iota(jnp.int32, (512, 64), 0) % 64 ==
                 lax.broadcasted_iota(jnp.int32, (512, 64), 1)).astype(F32) / 8.0
        ssrc = t0_ref[...][:, 128:136]
        z = ssrc + sd_ref[...][:, :8]
        wself = jnp.exp(jnp.maximum(z, 0.2 * z))
        ratios = []
        for g in range(4):
            ab, tb = arefs[g][...], trefs[g][...]
            ws2 = wself[:, 2 * g:2 * g + 2]
            num = ab[:, :128] + tb[:, :128] * jnp.dot(ws2, rep)
            den = jnp.dot(ab[:, 128 + 2 * g:130 + 2 * g] + ws2, rep)
            ratios.append(num / den)
        rat = jnp.concatenate(ratios, axis=1)
        out64 = jnp.dot(rat, mean8, preferred_element_type=F32)
        pvb = pv_ref[...]
        o = out64 + mm_ref[...] + pvb[0] + pvb[1]
        mu = o.mean(-1, keepdims=True)
        var = ((o - mu) ** 2).mean(-1, keepdims=True)
        h = jnp.maximum((o - mu) * lax.rsqrt(var + 1e-5) * pvb[2] + pvb[3], 0.0)
        t4_ref[...] = h * dv_ref[...][:, :1]
        h4_ref[...] = h

    return pl.pallas_call(
        body,
        grid=(N // RB,),
        in_specs=[_node_spec(GATW)] * 8 + [_node_spec(16), _node_spec(64),
                                           _node_spec(8), _full_spec(4, 64)],
        out_specs=[_node_spec(64), _node_spec(64)],
        out_shape=[_out(64), _out(64)],
    )(*accs, *t3s, sd3, mm3, dv8, pv)


def _tc5(p0, p1, h4, dv8, w4, wm4, bsum):
    def body(p0_ref, p1_ref, h4_ref, dv_ref, w4_ref, wm_ref, b_ref, o_ref):
        dinv = dv_ref[...][:, :1]
        h4b = h4_ref[...]
        gcn = (p0_ref[...] + p1_ref[...]) * dinv + h4b * dinv * dinv
        o_ref[...] = (jnp.dot(gcn, w4_ref[...], preferred_element_type=F32) +
                      jnp.dot(h4b, wm_ref[...], preferred_element_type=F32) +
                      b_ref[...])

    return pl.pallas_call(
        body,
        grid=(N // RB,),
        in_specs=[_node_spec(64), _node_spec(64), _node_spec(64),
                  _node_spec(8), _full_spec(64, 2), _full_spec(64, 2),
                  _full_spec(1, 2)],
        out_specs=_node_spec(2),
        out_shape=_out(2),
    )(p0, p1, h4, dv8, w4, wm4, bsum)


def _acat(a_src, a_dst, out_ch):
    c = 8 * out_ch
    hot = (jnp.arange(c)[:, None] // out_ch == jnp.arange(8)[None, :]
           ).astype(F32)
    return jnp.concatenate([a_src.reshape(-1)[:, None] * hot,
                            a_dst.reshape(-1)[:, None] * hot], axis=1)


def kernel(x, edge_index, W1, a_src1, a_dst1, b1, Wm1, bm1, g0, be0,
           W2, b2, Wm2, bm2, g1, be1, W3, a_src3, a_dst3, b3, Wm3, bm3,
           g2, be2, W4, b4, Wm4, bm4):
    npad = E_PAD - E
    src32 = edge_index[0].astype(jnp.int32)
    dst32 = edge_index[1].astype(jnp.int32)
    src = jnp.concatenate([src32,
                           jnp.full((npad,), N, jnp.int32)]).reshape(-1, B)
    dst = jnp.concatenate([dst32,
                           jnp.zeros((npad,), jnp.int32)]).reshape(-1, B)
    srcg = jnp.concatenate([src32,
                            jnp.zeros((npad,), jnp.int32)]).reshape(-1, 128)
    dstg = jnp.concatenate([dst32,
                            jnp.full((npad,), N, jnp.int32)]).reshape(-1, 128)

    dummy_gat = jnp.concatenate([jnp.zeros((128,), F32),
                                 jnp.full((16,), -1e30, F32)])[None]

    t0, t1, sd1, mm1 = _tc1(x, jnp.concatenate([W1, Wm1], axis=1),
                            _acat(a_src1, a_dst1, 32))
    table1 = jnp.concatenate([t0, dummy_gat, t1, dummy_gat], axis=0)
    acc1 = _gat_sc(table1, sd1, src, dst, lpg=4, loff=0, cph=2)
    t2, h2, mm2, dv8 = _tc2(acc1[:N], acc1[N_ACC:N_ACC + N], t0, t1, sd1, mm1,
                            jnp.stack([b1, bm1, g0, be0]),
                            jnp.concatenate([W2, Wm2], axis=1))

    p2 = _gcn_sc(t2, srcg, dstg, 128)
    t30, t31, t32, t33, sd3, mm3 = _tc3(p2[:N], p2[N_ACC:N_ACC + N], h2,
                                        mm2, dv8,
                                        jnp.stack([b2, bm2, g1, be1]),
                                        W3, _acat(a_src3, a_dst3, 64), Wm3)

    tableA = jnp.concatenate([t30, dummy_gat, t31, dummy_gat], axis=0)
    tableB = jnp.concatenate([t32, dummy_gat, t33, dummy_gat], axis=0)
    accA = _gat_sc(tableA, sd3, src, dst, lpg=2, loff=0, cph=4)
    accB = _gat_sc(tableB, sd3, src, dst, lpg=2, loff=4, cph=4)
    t4, h4 = _tc4((accA[:N], accA[N_ACC:N_ACC + N],
                   accB[:N], accB[N_ACC:N_ACC + N]),
                  (t30, t31, t32, t33), sd3, mm3, dv8,
                  jnp.stack([b3, bm3, g2, be2]))

    p4 = _gcn_sc(t4, srcg, dstg, 64)
    return _tc5(p4[:N], p4[N_ACC:N_ACC + N], h4, dv8, W4, Wm4,
                (b4 + bm4)[None])


# speedup vs baseline: 1.0713x; 1.0062x over previous
import functools

import jax
import jax.numpy as jnp
from jax import lax
from jax.experimental import pallas as pl
from jax.experimental.pallas import tpu as pltpu
from jax.experimental.pallas import tpu_sc as plsc

N = 10000
E = 160000
E_PAD = 163840
N_T = N + 1
NC, NS = 2, 16
N_ACC = 10112
RPT = N_ACC // NS
B = 64
GATW = 144


def _gat_sc(t0, t1, sdst, src2d, dst2d, *, lpg, loff, cph):
    ept = E_PAD // NS
    nbatch = ept // B
    hb = nbatch // 2
    mesh = plsc.VectorSubcoreMesh(core_axis_name="c", subcore_axis_name="s")

    @functools.partial(
        pl.kernel,
        out_type=jax.ShapeDtypeStruct((NC * N_ACC, GATW), jnp.float32),
        mesh=mesh,
        scratch_types=[
            pltpu.VMEM((hb, B), jnp.int32),
            pltpu.VMEM((hb, B), jnp.int32),
            pltpu.VMEM((B, GATW), jnp.float32),
            pltpu.VMEM((B, GATW), jnp.float32),
            pltpu.VMEM((B, 16), jnp.float32),
            pltpu.VMEM((B, 16), jnp.float32),
            pltpu.VMEM_SHARED((N_ACC, GATW), jnp.float32),
        ] + [pltpu.SemaphoreType.DMA] * 6,
        compiler_params=pltpu.CompilerParams(use_tc_tiling_on_sc=False),
    )
    def k(t0_hbm, t1_hbm, sdst_hbm, src_hbm, dst_hbm, out_hbm,
          srcb, dstb, rows0, rows1, sd0, sd1, acc,
          gs0, gs1, ds0, ds1, cs0, cs1):
        g = lax.axis_index("c")
        s = lax.axis_index("s")
        zeros16 = jnp.zeros((16,), jnp.float32)
        rows_ = (rows0, rows1)
        sd_ = (sd0, sd1)
        gs_ = (gs0, gs1)
        ds_ = (ds0, ds1)
        cs_ = (cs0, cs1)

        def zrow(i, c):
            for kk in range(GATW // 16):
                rows0[i, pl.ds(16 * kk, 16)] = zeros16
            return c
        lax.fori_loop(0, B, zrow, 0)
        base0 = s * RPT
        full, rem = RPT // B, RPT % B
        for j in range(full):
            pltpu.sync_copy(rows0, acc.at[pl.ds(base0 + j * B, B)])
        if rem:
            pltpu.sync_copy(rows0.at[pl.ds(0, rem)],
                            acc.at[pl.ds(base0 + full * B, rem)])
        plsc.subcore_barrier()

        def g_rows_start(sl, b):
            @pl.when(g == 0)
            def _t0():
                pltpu.make_async_copy(t0_hbm.at[srcb.at[b]],
                                      rows_[sl], gs_[sl]).start()

            @pl.when(g == 1)
            def _t1():
                pltpu.make_async_copy(t1_hbm.at[srcb.at[b]],
                                      rows_[sl], gs_[sl]).start()

        def g_rows_wait(sl, b):
            @pl.when(g == 0)
            def _t0():
                pltpu.make_async_copy(t0_hbm.at[srcb.at[b]],
                                      rows_[sl], gs_[sl]).wait()

            @pl.when(g == 1)
            def _t1():
                pltpu.make_async_copy(t1_hbm.at[srcb.at[b]],
                                      rows_[sl], gs_[sl]).wait()

        def g_sd(sl, b):
            return pltpu.make_async_copy(sdst_hbm.at[dstb.at[b]],
                                         sd_[sl], ds_[sl])

        def g_sc(sl, b):
            return pltpu.make_async_copy(rows_[sl], acc.at[dstb.at[b]],
                                         cs_[sl])

        def compute(sl):
            rows = rows_[sl]
            sd = sd_[sl]

            @plsc.parallel_loop(0, B, 1, unroll=4)
            def edge(e):
                sv = rows[e, pl.ds(128, 16)]
                dv = sd[e, :]
                z = sv + dv
                w = jnp.exp(jnp.maximum(z, 0.2 * z))
                rows[e, pl.ds(128, 16)] = w
                for hh in range(128 // (16 * cph)):
                    lane = g * lpg + (loff + hh)
                    wk = w.at[jnp.full((16,), lane, jnp.int32)].get(
                        mode="promise_in_bounds")
                    for p in range(cph):
                        kk = hh * cph + p
                        rows[e, pl.ds(16 * kk, 16)] = (
                            rows[e, pl.ds(16 * kk, 16)] * wk)

        def step(b, sl):
            other = 1 - sl
            nb = b + 1

            @pl.when(nb < hb)
            def _issue():
                @pl.when(nb >= 2)
                def _drain():
                    g_sc(other, 0).wait()
                g_rows_start(other, nb)
                g_sd(other, nb).start()

            g_rows_wait(sl, b)
            g_sd(sl, b).wait()
            compute(sl)
            g_sc(sl, b).start(add=True)

        def body(i, c):
            step(2 * i, 0)
            step(2 * i + 1, 1)
            return c

        for half in range(2):
            rbase = s * nbatch + half * hb
            pltpu.sync_copy(src_hbm.at[pl.ds(rbase, hb)], srcb)
            pltpu.sync_copy(dst_hbm.at[pl.ds(rbase, hb)], dstb)

            g_rows_start(0, 0)
            g_sd(0, 0).start()
            lax.fori_loop(0, hb // 2, body, 0)
            g_sc(0, 0).wait()
            g_sc(1, 0).wait()

        plsc.subcore_barrier()
        pltpu.sync_copy(acc.at[pl.ds(base0, RPT)],
                        out_hbm.at[pl.ds(g * N_ACC + base0, RPT)])

    return k(t0, t1, sdst, src2d, dst2d)


def _gcn_sc(table, src3d, dst3d, width):
    BG = 128
    ept = E_PAD // (NC * NS)
    nbatch = ept // BG
    nslot = 2
    mesh = plsc.VectorSubcoreMesh(core_axis_name="c", subcore_axis_name="s")

    @functools.partial(
        pl.kernel,
        out_type=jax.ShapeDtypeStruct((NC * N_ACC, width), jnp.float32),
        mesh=mesh,
        scratch_types=[
            pltpu.VMEM((nbatch, BG), jnp.int32),
            pltpu.VMEM((nbatch, BG), jnp.int32),
        ] + [pltpu.VMEM((BG, width), jnp.float32)] * nslot
          + [pltpu.VMEM_SHARED((N_ACC, width), jnp.float32)]
          + [pltpu.SemaphoreType.DMA] * (2 * nslot),
        compiler_params=pltpu.CompilerParams(use_tc_tiling_on_sc=False),
    )
    def k(table_hbm, src_hbm, dst_hbm, out_hbm, srcb, dstb, *scr):
        rows_ = scr[:nslot]
        acc = scr[nslot]
        gs_ = scr[nslot + 1:2 * nslot + 1]
        cs_ = scr[2 * nslot + 1:]
        g = lax.axis_index("c")
        s = lax.axis_index("s")
        wid = g * NS + s
        zeros16 = jnp.zeros((16,), jnp.float32)

        rb = wid * nbatch
        pltpu.sync_copy(src_hbm.at[pl.ds(rb, nbatch)], srcb)
        pltpu.sync_copy(dst_hbm.at[pl.ds(rb, nbatch)], dstb)

        zbuf = rows_[0]

        def zrow(i, c):
            for kk in range(width // 16):
                zbuf[i, pl.ds(16 * kk, 16)] = zeros16
            return c
        lax.fori_loop(0, BG, zrow, 0)
        base0 = s * RPT
        full, rem = RPT // BG, RPT % BG
        for j in range(full):
            pltpu.sync_copy(zbuf, acc.at[pl.ds(base0 + j * BG, BG)])
        if rem:
            pltpu.sync_copy(zbuf.at[pl.ds(0, rem)],
                            acc.at[pl.ds(base0 + full * BG, rem)])
        plsc.subcore_barrier()

        def g_rows(sl, b):
            return pltpu.make_async_copy(table_hbm.at[srcb.at[b]],
                                         rows_[sl], gs_[sl])

        def g_sc(sl, b):
            return pltpu.make_async_copy(rows_[sl], acc.at[dstb.at[b]],
                                         cs_[sl])

        def step(t, sl):
            other = 1 - sl
            nb = t + 1

            @pl.when(nb < nbatch)
            def _issue():
                @pl.when(nb >= 2)
                def _drain():
                    g_sc(other, 0).wait()
                g_rows(other, nb).start()

            g_rows(sl, t).wait()
            g_sc(sl, t).start(add=True)

        g_rows(0, 0).start()

        def body(i, c):
            step(2 * i, 0)
            step(2 * i + 1, 1)
            return c
        lax.fori_loop(0, nbatch // 2, body, 0)
        g_sc(0, 0).wait()
        g_sc(1, 0).wait()

        plsc.subcore_barrier()
        pltpu.sync_copy(acc.at[pl.ds(base0, RPT)],
                        out_hbm.at[pl.ds(g * N_ACC + base0, RPT)])

    return k(table, src3d, dst3d)


RB = 1000
F32 = jnp.float32


def _node_spec(width):
    return pl.BlockSpec((RB, width), lambda i: (i, 0))


def _full_spec(r, c):
    return pl.BlockSpec((r, c), lambda i: (0, 0))


def _out(width):
    return jax.ShapeDtypeStruct((N, width), F32)


def _tc1(x, wcat, acat):
    def body(x_ref, w_ref, a_ref, t0_ref, t1_ref, sd_ref, mm_ref):
        xb = x_ref[...]
        hcat = jnp.dot(xb, w_ref[...], preferred_element_type=F32)
        h1 = hcat[:, :256]
        mm_ref[...] = hcat[:, 256:]
        sall = jnp.dot(h1, a_ref[...], preferred_element_type=F32)
        zp = jnp.zeros((RB, 8), F32)
        t0_ref[...] = jnp.concatenate([h1[:, :128], sall[:, :8], zp], axis=1)
        t1_ref[...] = jnp.concatenate([h1[:, 128:], sall[:, :8], zp], axis=1)
        sd_ref[...] = jnp.concatenate([sall[:, 8:], zp], axis=1)

    return pl.pallas_call(
        body,
        grid=(N // RB,),
        in_specs=[_node_spec(1024), _full_spec(1024, 512), _full_spec(256, 16)],
        out_specs=[_node_spec(GATW), _node_spec(GATW), _node_spec(16),
                   _node_spec(256)],
        out_shape=[_out(GATW), _out(GATW), _out(16), _out(256)],
    )(x, wcat, acat)


def _tc2(a0, a1, t0, t1, sd1, mm1, pv, w2cat):
    def body(a0_ref, a1_ref, t0_ref, t1_ref, sd_ref, mm_ref, pv_ref, w_ref,
             t2_ref, h2_ref, mm2_ref, dv_ref):
        a0b, a1b = a0_ref[...], a1_ref[...]
        t0b, t1b = t0_ref[...], t1_ref[...]
        rep = (lax.broadcasted_iota(jnp.int32, (4, 128), 1) // 32 ==
               lax.broadcasted_iota(jnp.int32, (4, 128), 0)).astype(F32)
        ssrc = t0b[:, 128:136]
        sdst = sd_ref[...][:, :8]
        z = ssrc + sdst
        wself = jnp.exp(jnp.maximum(z, 0.2 * z))
        deg = a0b[:, 136:137] + 1.0
        dinv = lax.rsqrt(deg)
        num0 = a0b[:, :128] + t0b[:, :128] * jnp.dot(wself[:, :4], rep)
        den0 = jnp.dot(a0b[:, 128:132] + wself[:, :4], rep)
        num1 = a1b[:, :128] + t1b[:, :128] * jnp.dot(wself[:, 4:], rep)
        den1 = jnp.dot(a1b[:, 132:136] + wself[:, 4:], rep)
        gat = jnp.concatenate([num0 / den0, num1 / den1], axis=1)
        pvb = pv_ref[...]
        o = gat + mm_ref[...] + pvb[0] + pvb[1]
        mu = o.mean(-1, keepdims=True)
        var = ((o - mu) ** 2).mean(-1, keepdims=True)
        h = jnp.maximum((o - mu) * lax.rsqrt(var + 1e-5) * pvb[2] + pvb[3], 0.0)
        hcat = jnp.dot(h, w_ref[...], preferred_element_type=F32)
        h2 = hcat[:, :128]
        h2_ref[...] = h2
        mm2_ref[...] = hcat[:, 128:]
        t2_ref[...] = h2 * dinv
        dv_ref[...] = jnp.broadcast_to(dinv, (RB, 8))

    return pl.pallas_call(
        body,
        grid=(N // RB,),
        in_specs=[_node_spec(GATW), _node_spec(GATW), _node_spec(GATW),
                  _node_spec(GATW), _node_spec(16), _node_spec(256),
                  _full_spec(4, 256), _full_spec(256, 256)],
        out_specs=[_node_spec(128), _node_spec(128), _node_spec(128),
                   _node_spec(8)],
        out_shape=[_out(128), _out(128), _out(128), _out(8)],
    )(a0, a1, t0, t1, sd1, mm1, pv, w2cat)


def _tc3(p0, p1, h2, mm2, dv8, pv, w3, acat3, wm3):
    def body(p0_ref, p1_ref, h2_ref, mm2_ref, dv_ref, pv_ref, w3_ref, a_ref,
             wm_ref, t30_ref, t31_ref, t32_ref, t33_ref, sd_ref, mm3_ref):
        dinv = dv_ref[...][:, :1]
        gcn = (p0_ref[...] + p1_ref[...]) * dinv + h2_ref[...] * dinv * dinv
        pvb = pv_ref[...]
        o = gcn + mm2_ref[...] + pvb[0] + pvb[1]
        mu = o.mean(-1, keepdims=True)
        var = ((o - mu) ** 2).mean(-1, keepdims=True)
        h = jnp.maximum((o - mu) * lax.rsqrt(var + 1e-5) * pvb[2] + pvb[3], 0.0)
        h3 = jnp.dot(h, w3_ref[...], preferred_element_type=F32)
        sall = jnp.dot(h3, a_ref[...], preferred_element_type=F32)
        mm3_ref[...] = jnp.dot(h, wm_ref[...], preferred_element_type=F32)
        zp = jnp.zeros((RB, 8), F32)
        for g, tref in enumerate((t30_ref, t31_ref, t32_ref, t33_ref)):
            tref[...] = jnp.concatenate(
                [h3[:, 128 * g:128 * (g + 1)], sall[:, :8], zp], axis=1)
        sd_ref[...] = jnp.concatenate([sall[:, 8:], zp], axis=1)

    return pl.pallas_call(
        body,
        grid=(N // RB,),
        in_specs=[_node_spec(128), _node_spec(128), _node_spec(128),
                  _node_spec(128), _node_spec(8), _full_spec(4, 128),
                  _full_spec(128, 512), _full_spec(512, 16),
                  _full_spec(128, 64)],
        out_specs=[_node_spec(GATW)] * 4 + [_node_spec(16), _node_spec(64)],
        out_shape=[_out(GATW)] * 4 + [_out(16), _out(64)],
    )(p0, p1, h2, mm2, dv8, pv, w3, acat3, wm3)


def _tc4(accs, t3s, sd3, mm3, dv8, pv):
    def body(a0_ref, a1_ref, a2_ref, a3_ref, t0_ref, t1_ref, t2_ref, t3_ref,
             sd_ref, mm_ref, dv_ref, pv_ref, t4_ref, h4_ref):
        arefs = (a0_ref, a1_ref, a2_ref, a3_ref)
        trefs = (t0_ref, t1_ref, t2_ref, t3_ref)
        rep = (lax.broadcasted_iota(jnp.int32, (2, 128), 1) // 64 ==
               lax.broadcasted_iota(jnp.int32, (2, 128), 0)).astype(F32)
        mean8 = (lax.broadcasted_iota(jnp.int32, (512, 64), 0) % 64 ==
                 lax.broadcasted_iota(jnp.int32, (512, 64), 1)).astype(F32) / 8.0
        ssrc = t0_ref[...][:, 128:136]
        z = ssrc + sd_ref[...][:, :8]
        wself = jnp.exp(jnp.maximum(z, 0.2 * z))
        ratios = []
        for g in range(4):
            ab, tb = arefs[g][...], trefs[g][...]
            ws2 = wself[:, 2 * g:2 * g + 2]
            num = ab[:, :128] + tb[:, :128] * jnp.dot(ws2, rep)
            den = jnp.dot(ab[:, 128 + 2 * g:130 + 2 * g] + ws2, rep)
            ratios.append(num / den)
        rat = jnp.concatenate(ratios, axis=1)
        out64 = jnp.dot(rat, mean8, preferred_element_type=F32)
        pvb = pv_ref[...]
        o = out64 + mm_ref[...] + pvb[0] + pvb[1]
        mu = o.mean(-1, keepdims=True)
        var = ((o - mu) ** 2).mean(-1, keepdims=True)
        h = jnp.maximum((o - mu) * lax.rsqrt(var + 1e-5) * pvb[2] + pvb[3], 0.0)
        t4_ref[...] = h * dv_ref[...][:, :1]
        h4_ref[...] = h

    return pl.pallas_call(
        body,
        grid=(N // RB,),
        in_specs=[_node_spec(GATW)] * 8 + [_node_spec(16), _node_spec(64),
                                           _node_spec(8), _full_spec(4, 64)],
        out_specs=[_node_spec(64), _node_spec(64)],
        out_shape=[_out(64), _out(64)],
    )(*accs, *t3s, sd3, mm3, dv8, pv)


def _tc5(p0, p1, h4, dv8, w4, wm4, bsum):
    def body(p0_ref, p1_ref, h4_ref, dv_ref, w4_ref, wm_ref, b_ref, o_ref):
        dinv = dv_ref[...][:, :1]
        h4b = h4_ref[...]
        gcn = (p0_ref[...] + p1_ref[...]) * dinv + h4b * dinv * dinv
        o_ref[...] = (jnp.dot(gcn, w4_ref[...], preferred_element_type=F32) +
                      jnp.dot(h4b, wm_ref[...], preferred_element_type=F32) +
                      b_ref[...])

    return pl.pallas_call(
        body,
        grid=(N // RB,),
        in_specs=[_node_spec(64), _node_spec(64), _node_spec(64),
                  _node_spec(8), _full_spec(64, 2), _full_spec(64, 2),
                  _full_spec(1, 2)],
        out_specs=_node_spec(2),
        out_shape=_out(2),
    )(p0, p1, h4, dv8, w4, wm4, bsum)


def _acat(a_src, a_dst, out_ch):
    c = 8 * out_ch
    hot = (jnp.arange(c)[:, None] // out_ch == jnp.arange(8)[None, :]
           ).astype(F32)
    return jnp.concatenate([a_src.reshape(-1)[:, None] * hot,
                            a_dst.reshape(-1)[:, None] * hot], axis=1)


def kernel(x, edge_index, W1, a_src1, a_dst1, b1, Wm1, bm1, g0, be0,
           W2, b2, Wm2, bm2, g1, be1, W3, a_src3, a_dst3, b3, Wm3, bm3,
           g2, be2, W4, b4, Wm4, bm4):
    npad = E_PAD - E
    srcp = jnp.concatenate([edge_index[0].astype(jnp.int32),
                            jnp.zeros((npad,), jnp.int32)])
    dstp = jnp.concatenate([edge_index[1].astype(jnp.int32),
                            jnp.full((npad,), N, jnp.int32)])
    src, dst = srcp.reshape(-1, B), dstp.reshape(-1, B)
    srcg, dstg = srcp.reshape(-1, 128), dstp.reshape(-1, 128)
    sdpad = jnp.zeros((N_ACC - N, 16), F32)

    t0, t1, sd1, mm1 = _tc1(x, jnp.concatenate([W1, Wm1], axis=1),
                            _acat(a_src1, a_dst1, 32))
    sd1e = jnp.concatenate([sd1, sdpad], axis=0)
    acc1 = _gat_sc(t0, t1, sd1e, src, dst, lpg=4, loff=0, cph=2)
    t2, h2, mm2, dv8 = _tc2(acc1[:N], acc1[N_ACC:N_ACC + N], t0, t1, sd1, mm1,
                            jnp.stack([b1, bm1, g0, be0]),
                            jnp.concatenate([W2, Wm2], axis=1))

    p2 = _gcn_sc(t2, srcg, dstg, 128)
    t30, t31, t32, t33, sd3, mm3 = _tc3(p2[:N], p2[N_ACC:N_ACC + N], h2,
                                        mm2, dv8,
                                        jnp.stack([b2, bm2, g1, be1]),
                                        W3, _acat(a_src3, a_dst3, 64), Wm3)

    sd3e = jnp.concatenate([sd3, sdpad], axis=0)
    accA = _gat_sc(t30, t31, sd3e, src, dst, lpg=2, loff=0, cph=4)
    accB = _gat_sc(t32, t33, sd3e, src, dst, lpg=2, loff=4, cph=4)
    t4, h4 = _tc4((accA[:N], accA[N_ACC:N_ACC + N],
                   accB[:N], accB[N_ACC:N_ACC + N]),
                  (t30, t31, t32, t33), sd3, mm3, dv8,
                  jnp.stack([b3, bm3, g2, be2]))

    p4 = _gcn_sc(t4, srcg, dstg, 64)
    return _tc5(p4[:N], p4[N_ACC:N_ACC + N], h4, dv8, W4, Wm4,
                (b4 + bm4)[None])


# speedup vs baseline: 1.0966x; 1.0236x over previous
import functools

import jax
import jax.numpy as jnp
from jax import lax
from jax.experimental import pallas as pl
from jax.experimental.pallas import tpu as pltpu
from jax.experimental.pallas import tpu_sc as plsc

N = 10000
E = 160000
E_PAD = 163840
N_T = N + 1
NC, NS = 2, 16
N_ACC = 10112
RPT = N_ACC // NS
B = 64
GATW = 144


def _gat_sc(t0, t1, sdst, src2d, dst2d, zrows, *, lpg, loff, cph):
    ept = E_PAD // NS
    nbatch = ept // B
    hb = nbatch // 2
    mesh = plsc.VectorSubcoreMesh(core_axis_name="c", subcore_axis_name="s")

    @functools.partial(
        pl.kernel,
        out_type=jax.ShapeDtypeStruct((NC * N_ACC, GATW), jnp.float32),
        mesh=mesh,
        scratch_types=[
            pltpu.VMEM((hb, B), jnp.int32),
            pltpu.VMEM((hb, B), jnp.int32),
            pltpu.VMEM((B, GATW), jnp.float32),
            pltpu.VMEM((B, GATW), jnp.float32),
            pltpu.VMEM((B, 16), jnp.float32),
            pltpu.VMEM((B, 16), jnp.float32),
            pltpu.VMEM_SHARED((N_ACC, GATW), jnp.float32),
        ] + [pltpu.SemaphoreType.DMA] * 6,
        compiler_params=pltpu.CompilerParams(use_tc_tiling_on_sc=False),
    )
    def k(t0_hbm, t1_hbm, sdst_hbm, src_hbm, dst_hbm, z_hbm, out_hbm,
          srcb, dstb, rows0, rows1, sd0, sd1, acc,
          gs0, gs1, ds0, ds1, cs0, cs1):
        g = lax.axis_index("c")
        s = lax.axis_index("s")
        rows_ = (rows0, rows1)
        sd_ = (sd0, sd1)
        gs_ = (gs0, gs1)
        ds_ = (ds0, ds1)
        cs_ = (cs0, cs1)

        base0 = s * RPT
        pltpu.sync_copy(z_hbm, acc.at[pl.ds(base0, RPT)])
        plsc.subcore_barrier()

        def g_rows_start(sl, b):
            @pl.when(g == 0)
            def _t0():
                pltpu.make_async_copy(t0_hbm.at[srcb.at[b]],
                                      rows_[sl], gs_[sl]).start()

            @pl.when(g == 1)
            def _t1():
                pltpu.make_async_copy(t1_hbm.at[srcb.at[b]],
                                      rows_[sl], gs_[sl]).start()

        def g_rows_wait(sl, b):
            @pl.when(g == 0)
            def _t0():
                pltpu.make_async_copy(t0_hbm.at[srcb.at[b]],
                                      rows_[sl], gs_[sl]).wait()

            @pl.when(g == 1)
            def _t1():
                pltpu.make_async_copy(t1_hbm.at[srcb.at[b]],
                                      rows_[sl], gs_[sl]).wait()

        def g_sd(sl, b):
            return pltpu.make_async_copy(sdst_hbm.at[dstb.at[b]],
                                         sd_[sl], ds_[sl])

        def g_sc(sl, b):
            return pltpu.make_async_copy(rows_[sl], acc.at[dstb.at[b]],
                                         cs_[sl])

        def compute(sl):
            rows = rows_[sl]
            sd = sd_[sl]

            @plsc.parallel_loop(0, B, 1, unroll=4)
            def edge(e):
                sv = rows[e, pl.ds(128, 16)]
                dv = sd[e, :]
                z = sv + dv
                w = jnp.exp(jnp.maximum(z, 0.2 * z))
                rows[e, pl.ds(128, 16)] = w
                for hh in range(128 // (16 * cph)):
                    lane = g * lpg + (loff + hh)
                    wk = w.at[jnp.full((16,), lane, jnp.int32)].get(
                        mode="promise_in_bounds")
                    for p in range(cph):
                        kk = hh * cph + p
                        rows[e, pl.ds(16 * kk, 16)] = (
                            rows[e, pl.ds(16 * kk, 16)] * wk)

        def step(b, sl):
            other = 1 - sl
            nb = b + 1

            @pl.when(nb < hb)
            def _issue():
                @pl.when(nb >= 2)
                def _drain():
                    g_sc(other, 0).wait()
                g_rows_start(other, nb)
                g_sd(other, nb).start()

            g_rows_wait(sl, b)
            g_sd(sl, b).wait()
            compute(sl)
            g_sc(sl, b).start(add=True)

        def body(i, c):
            step(2 * i, 0)
            step(2 * i + 1, 1)
            return c

        for half in range(2):
            rbase = s * nbatch + half * hb
            pltpu.sync_copy(src_hbm.at[pl.ds(rbase, hb)], srcb)
            pltpu.sync_copy(dst_hbm.at[pl.ds(rbase, hb)], dstb)

            g_rows_start(0, 0)
            g_sd(0, 0).start()
            lax.fori_loop(0, hb // 2, body, 0)
            g_sc(0, 0).wait()
            g_sc(1, 0).wait()

        plsc.subcore_barrier()
        pltpu.sync_copy(acc.at[pl.ds(base0, RPT)],
                        out_hbm.at[pl.ds(g * N_ACC + base0, RPT)])

    return k(t0, t1, sdst, src2d, dst2d, zrows)


def _gcn_sc(table, src3d, dst3d, zrows, width):
    BG = 128
    ept = E_PAD // (NC * NS)
    nbatch = ept // BG
    nslot = 2
    mesh = plsc.VectorSubcoreMesh(core_axis_name="c", subcore_axis_name="s")

    @functools.partial(
        pl.kernel,
        out_type=jax.ShapeDtypeStruct((NC * N_ACC, width), jnp.float32),
        mesh=mesh,
        scratch_types=[
            pltpu.VMEM((nbatch, BG), jnp.int32),
            pltpu.VMEM((nbatch, BG), jnp.int32),
        ] + [pltpu.VMEM((BG, width), jnp.float32)] * nslot
          + [pltpu.VMEM_SHARED((N_ACC, width), jnp.float32)]
          + [pltpu.SemaphoreType.DMA] * (2 * nslot),
        compiler_params=pltpu.CompilerParams(use_tc_tiling_on_sc=False),
    )
    def k(table_hbm, src_hbm, dst_hbm, z_hbm, out_hbm, srcb, dstb, *scr):
        rows_ = scr[:nslot]
        acc = scr[nslot]
        gs_ = scr[nslot + 1:2 * nslot + 1]
        cs_ = scr[2 * nslot + 1:]
        g = lax.axis_index("c")
        s = lax.axis_index("s")
        wid = g * NS + s

        rb = wid * nbatch
        pltpu.sync_copy(src_hbm.at[pl.ds(rb, nbatch)], srcb)
        pltpu.sync_copy(dst_hbm.at[pl.ds(rb, nbatch)], dstb)

        base0 = s * RPT
        pltpu.sync_copy(z_hbm, acc.at[pl.ds(base0, RPT)])
        plsc.subcore_barrier()

        def g_rows(sl, b):
            return pltpu.make_async_copy(table_hbm.at[srcb.at[b]],
                                         rows_[sl], gs_[sl])

        def g_sc(sl, b):
            return pltpu.make_async_copy(rows_[sl], acc.at[dstb.at[b]],
                                         cs_[sl])

        def step(t, sl):
            other = 1 - sl
            nb = t + 1

            @pl.when(nb < nbatch)
            def _issue():
                @pl.when(nb >= 2)
                def _drain():
                    g_sc(other, 0).wait()
                g_rows(other, nb).start()

            g_rows(sl, t).wait()
            g_sc(sl, t).start(add=True)

        g_rows(0, 0).start()

        def body(i, c):
            step(2 * i, 0)
            step(2 * i + 1, 1)
            return c
        lax.fori_loop(0, nbatch // 2, body, 0)
        g_sc(0, 0).wait()
        g_sc(1, 0).wait()

        plsc.subcore_barrier()
        pltpu.sync_copy(acc.at[pl.ds(base0, RPT)],
                        out_hbm.at[pl.ds(g * N_ACC + base0, RPT)])

    return k(table, src3d, dst3d, zrows)


RB = 1000
F32 = jnp.float32


def _node_spec(width):
    return pl.BlockSpec((RB, width), lambda i: (i, 0))


def _full_spec(r, c):
    return pl.BlockSpec((r, c), lambda i: (0, 0))


def _out(width):
    return jax.ShapeDtypeStruct((N, width), F32)


def _tc1(x, wcat, acat):
    def body(x_ref, w_ref, a_ref, t0_ref, t1_ref, sd_ref, mm_ref):
        xb = x_ref[...]
        hcat = jnp.dot(xb, w_ref[...], preferred_element_type=F32)
        h1 = hcat[:, :256]
        mm_ref[...] = hcat[:, 256:]
        sall = jnp.dot(h1, a_ref[...], preferred_element_type=F32)
        zp = jnp.zeros((RB, 8), F32)
        t0_ref[...] = jnp.concatenate([h1[:, :128], sall[:, :8], zp], axis=1)
        t1_ref[...] = jnp.concatenate([h1[:, 128:], sall[:, :8], zp], axis=1)
        sd_ref[...] = jnp.concatenate([sall[:, 8:], zp], axis=1)

    return pl.pallas_call(
        body,
        grid=(N // RB,),
        in_specs=[_node_spec(1024), _full_spec(1024, 512), _full_spec(256, 16)],
        out_specs=[_node_spec(GATW), _node_spec(GATW), _node_spec(16),
                   _node_spec(256)],
        out_shape=[_out(GATW), _out(GATW), _out(16), _out(256)],
    )(x, wcat, acat)


def _tc2(a0, a1, t0, t1, sd1, mm1, pv, w2cat):
    def body(a0_ref, a1_ref, t0_ref, t1_ref, sd_ref, mm_ref, pv_ref, w_ref,
             t2_ref, h2_ref, mm2_ref, dv_ref):
        a0b, a1b = a0_ref[...], a1_ref[...]
        t0b, t1b = t0_ref[...], t1_ref[...]
        rep = (lax.broadcasted_iota(jnp.int32, (4, 128), 1) // 32 ==
               lax.broadcasted_iota(jnp.int32, (4, 128), 0)).astype(F32)
        ssrc = t0b[:, 128:136]
        sdst = sd_ref[...][:, :8]
        z = ssrc + sdst
        wself = jnp.exp(jnp.maximum(z, 0.2 * z))
        deg = a0b[:, 136:137] + 1.0
        dinv = lax.rsqrt(deg)
        num0 = a0b[:, :128] + t0b[:, :128] * jnp.dot(wself[:, :4], rep)
        den0 = jnp.dot(a0b[:, 128:132] + wself[:, :4], rep)
        num1 = a1b[:, :128] + t1b[:, :128] * jnp.dot(wself[:, 4:], rep)
        den1 = jnp.dot(a1b[:, 132:136] + wself[:, 4:], rep)
        gat = jnp.concatenate([num0 / den0, num1 / den1], axis=1)
        pvb = pv_ref[...]
        o = gat + mm_ref[...] + pvb[0] + pvb[1]
        mu = o.mean(-1, keepdims=True)
        var = ((o - mu) ** 2).mean(-1, keepdims=True)
        h = jnp.maximum((o - mu) * lax.rsqrt(var + 1e-5) * pvb[2] + pvb[3], 0.0)
        hcat = jnp.dot(h, w_ref[...], preferred_element_type=F32)
        h2 = hcat[:, :128]
        h2_ref[...] = h2
        mm2_ref[...] = hcat[:, 128:]
        t2_ref[...] = h2 * dinv
        dv_ref[...] = jnp.broadcast_to(dinv, (RB, 8))

    return pl.pallas_call(
        body,
        grid=(N // RB,),
        in_specs=[_node_spec(GATW), _node_spec(GATW), _node_spec(GATW),
                  _node_spec(GATW), _node_spec(16), _node_spec(256),
                  _full_spec(4, 256), _full_spec(256, 256)],
        out_specs=[_node_spec(128), _node_spec(128), _node_spec(128),
                   _node_spec(8)],
        out_shape=[_out(128), _out(128), _out(128), _out(8)],
    )(a0, a1, t0, t1, sd1, mm1, pv, w2cat)


def _tc3(p0, p1, h2, mm2, dv8, pv, w3, acat3, wm3):
    def body(p0_ref, p1_ref, h2_ref, mm2_ref, dv_ref, pv_ref, w3_ref, a_ref,
             wm_ref, t30_ref, t31_ref, t32_ref, t33_ref, sd_ref, mm3_ref):
        dinv = dv_ref[...][:, :1]
        gcn = (p0_ref[...] + p1_ref[...]) * dinv + h2_ref[...] * dinv * dinv
        pvb = pv_ref[...]
        o = gcn + mm2_ref[...] + pvb[0] + pvb[1]
        mu = o.mean(-1, keepdims=True)
        var = ((o - mu) ** 2).mean(-1, keepdims=True)
        h = jnp.maximum((o - mu) * lax.rsqrt(var + 1e-5) * pvb[2] + pvb[3], 0.0)
        h3 = jnp.dot(h, w3_ref[...], preferred_element_type=F32)
        sall = jnp.dot(h3, a_ref[...], preferred_element_type=F32)
        mm3_ref[...] = jnp.dot(h, wm_ref[...], preferred_element_type=F32)
        zp = jnp.zeros((RB, 8), F32)
        for g, tref in enumerate((t30_ref, t31_ref, t32_ref, t33_ref)):
            tref[...] = jnp.concatenate(
                [h3[:, 128 * g:128 * (g + 1)], sall[:, :8], zp], axis=1)
        sd_ref[...] = jnp.concatenate([sall[:, 8:], zp], axis=1)

    return pl.pallas_call(
        body,
        grid=(N // RB,),
        in_specs=[_node_spec(128), _node_spec(128), _node_spec(128),
                  _node_spec(128), _node_spec(8), _full_spec(4, 128),
                  _full_spec(128, 512), _full_spec(512, 16),
                  _full_spec(128, 64)],
        out_specs=[_node_spec(GATW)] * 4 + [_node_spec(16), _node_spec(64)],
        out_shape=[_out(GATW)] * 4 + [_out(16), _out(64)],
    )(p0, p1, h2, mm2, dv8, pv, w3, acat3, wm3)


def _tc4(accs, t3s, sd3, mm3, dv8, pv):
    def body(a0_ref, a1_ref, a2_ref, a3_ref, t0_ref, t1_ref, t2_ref, t3_ref,
             sd_ref, mm_ref, dv_ref, pv_ref, t4_ref, h4_ref):
        arefs = (a0_ref, a1_ref, a2_ref, a3_ref)
        trefs = (t0_ref, t1_ref, t2_ref, t3_ref)
        rep = (lax.broadcasted_iota(jnp.int32, (2, 128), 1) // 64 ==
               lax.broadcasted_iota(jnp.int32, (2, 128), 0)).astype(F32)
        mean8 = (lax.broadcasted_iota(jnp.int32, (512, 64), 0) % 64 ==
                 lax.broadcasted_iota(jnp.int32, (512, 64), 1)).astype(F32) / 8.0
        ssrc = t0_ref[...][:, 128:136]
        z = ssrc + sd_ref[...][:, :8]
        wself = jnp.exp(jnp.maximum(z, 0.2 * z))
        ratios = []
        for g in range(4):
            ab, tb = arefs[g][...], trefs[g][...]
            ws2 = wself[:, 2 * g:2 * g + 2]
            num = ab[:, :128] + tb[:, :128] * jnp.dot(ws2, rep)
            den = jnp.dot(ab[:, 128 + 2 * g:130 + 2 * g] + ws2, rep)
            ratios.append(num / den)
        rat = jnp.concatenate(ratios, axis=1)
        out64 = jnp.dot(rat, mean8, preferred_element_type=F32)
        pvb = pv_ref[...]
        o = out64 + mm_ref[...] + pvb[0] + pvb[1]
        mu = o.mean(-1, keepdims=True)
        var = ((o - mu) ** 2).mean(-1, keepdims=True)
        h = jnp.maximum((o - mu) * lax.rsqrt(var + 1e-5) * pvb[2] + pvb[3], 0.0)
        t4_ref[...] = h * dv_ref[...][:, :1]
        h4_ref[...] = h

    return pl.pallas_call(
        body,
        grid=(N // RB,),
        in_specs=[_node_spec(GATW)] * 8 + [_node_spec(16), _node_spec(64),
                                           _node_spec(8), _full_spec(4, 64)],
        out_specs=[_node_spec(64), _node_spec(64)],
        out_shape=[_out(64), _out(64)],
    )(*accs, *t3s, sd3, mm3, dv8, pv)


def _tc5(p0, p1, h4, dv8, w4, wm4, bsum):
    def body(p0_ref, p1_ref, h4_ref, dv_ref, w4_ref, wm_ref, b_ref, o_ref):
        dinv = dv_ref[...][:, :1]
        h4b = h4_ref[...]
        gcn = (p0_ref[...] + p1_ref[...]) * dinv + h4b * dinv * dinv
        o_ref[...] = (jnp.dot(gcn, w4_ref[...], preferred_element_type=F32) +
                      jnp.dot(h4b, wm_ref[...], preferred_element_type=F32) +
                      b_ref[...])

    return pl.pallas_call(
        body,
        grid=(N // RB,),
        in_specs=[_node_spec(64), _node_spec(64), _node_spec(64),
                  _node_spec(8), _full_spec(64, 2), _full_spec(64, 2),
                  _full_spec(1, 2)],
        out_specs=_node_spec(2),
        out_shape=_out(2),
    )(p0, p1, h4, dv8, w4, wm4, bsum)


def _acat(a_src, a_dst, out_ch):
    c = 8 * out_ch
    hot = (jnp.arange(c)[:, None] // out_ch == jnp.arange(8)[None, :]
           ).astype(F32)
    return jnp.concatenate([a_src.reshape(-1)[:, None] * hot,
                            a_dst.reshape(-1)[:, None] * hot], axis=1)


def kernel(x, edge_index, W1, a_src1, a_dst1, b1, Wm1, bm1, g0, be0,
           W2, b2, Wm2, bm2, g1, be1, W3, a_src3, a_dst3, b3, Wm3, bm3,
           g2, be2, W4, b4, Wm4, bm4):
    npad = E_PAD - E
    srcp = jnp.concatenate([edge_index[0].astype(jnp.int32),
                            jnp.zeros((npad,), jnp.int32)])
    dstp = jnp.concatenate([edge_index[1].astype(jnp.int32),
                            jnp.full((npad,), N, jnp.int32)])
    src, dst = srcp.reshape(-1, B), dstp.reshape(-1, B)
    srcg, dstg = srcp.reshape(-1, 128), dstp.reshape(-1, 128)
    sdpad = jnp.zeros((N_ACC - N, 16), F32)
    zgat = jnp.zeros((RPT, GATW), F32)
    z128 = jnp.zeros((RPT, 128), F32)
    z64 = jnp.zeros((RPT, 64), F32)

    t0, t1, sd1, mm1 = _tc1(x, jnp.concatenate([W1, Wm1], axis=1),
                            _acat(a_src1, a_dst1, 32))
    sd1e = jnp.concatenate([sd1, sdpad], axis=0)
    acc1 = _gat_sc(t0, t1, sd1e, src, dst, zgat, lpg=4, loff=0, cph=2)
    t2, h2, mm2, dv8 = _tc2(acc1[:N], acc1[N_ACC:N_ACC + N], t0, t1, sd1, mm1,
                            jnp.stack([b1, bm1, g0, be0]),
                            jnp.concatenate([W2, Wm2], axis=1))

    p2 = _gcn_sc(t2, srcg, dstg, z128, 128)
    t30, t31, t32, t33, sd3, mm3 = _tc3(p2[:N], p2[N_ACC:N_ACC + N], h2,
                                        mm2, dv8,
                                        jnp.stack([b2, bm2, g1, be1]),
                                        W3, _acat(a_src3, a_dst3, 64), Wm3)

    sd3e = jnp.concatenate([sd3, sdpad], axis=0)
    accA = _gat_sc(t30, t31, sd3e, src, dst, zgat, lpg=2, loff=0, cph=4)
    accB = _gat_sc(t32, t33, sd3e, src, dst, zgat, lpg=2, loff=4, cph=4)
    t4, h4 = _tc4((accA[:N], accA[N_ACC:N_ACC + N],
                   accB[:N], accB[N_ACC:N_ACC + N]),
                  (t30, t31, t32, t33), sd3, mm3, dv8,
                  jnp.stack([b3, bm3, g2, be2]))

    p4 = _gcn_sc(t4, srcg, dstg, z64, 64)
    return _tc5(p4[:N], p4[N_ACC:N_ACC + N], h4, dv8, W4, Wm4,
                (b4 + bm4)[None])


# speedup vs baseline: 1.0984x; 1.0016x over previous
import functools

import jax
import jax.numpy as jnp
from jax import lax
from jax.experimental import pallas as pl
from jax.experimental.pallas import tpu as pltpu
from jax.experimental.pallas import tpu_sc as plsc

N = 10000
E = 160000
E_PAD = 163840
N_T = N + 1
NC, NS = 2, 16
N_ACC = 10112
RPT = N_ACC // NS
B = 64
GATW = 144


def _gat_sc(t0, t1, sdst, src2d, dst2d, zrows, *, lpg, loff, cph):
    ept = E_PAD // NS
    nbatch = ept // B
    hb = nbatch // 2
    mesh = plsc.VectorSubcoreMesh(core_axis_name="c", subcore_axis_name="s")

    @functools.partial(
        pl.kernel,
        out_type=jax.ShapeDtypeStruct((NC * N_ACC, GATW), jnp.float32),
        mesh=mesh,
        scratch_types=[
            pltpu.VMEM((hb, B), jnp.int32),
            pltpu.VMEM((hb, B), jnp.int32),
            pltpu.VMEM((B, GATW), jnp.float32),
            pltpu.VMEM((B, GATW), jnp.float32),
            pltpu.VMEM((B, 16), jnp.float32),
            pltpu.VMEM((B, 16), jnp.float32),
            pltpu.VMEM_SHARED((N_ACC, GATW), jnp.float32),
        ] + [pltpu.SemaphoreType.DMA] * 6,
        compiler_params=pltpu.CompilerParams(use_tc_tiling_on_sc=False),
    )
    def k(t0_hbm, t1_hbm, sdst_hbm, src_hbm, dst_hbm, z_hbm, out_hbm,
          srcb, dstb, rows0, rows1, sd0, sd1, acc,
          gs0, gs1, ds0, ds1, cs0, cs1):
        g = lax.axis_index("c")
        s = lax.axis_index("s")
        rows_ = (rows0, rows1)
        sd_ = (sd0, sd1)
        gs_ = (gs0, gs1)
        ds_ = (ds0, ds1)
        cs_ = (cs0, cs1)

        def g_rows_start(sl, b):
            @pl.when(g == 0)
            def _t0():
                pltpu.make_async_copy(t0_hbm.at[srcb.at[b]],
                                      rows_[sl], gs_[sl]).start()

            @pl.when(g == 1)
            def _t1():
                pltpu.make_async_copy(t1_hbm.at[srcb.at[b]],
                                      rows_[sl], gs_[sl]).start()

        def g_rows_wait(sl, b):
            @pl.when(g == 0)
            def _t0():
                pltpu.make_async_copy(t0_hbm.at[srcb.at[b]],
                                      rows_[sl], gs_[sl]).wait()

            @pl.when(g == 1)
            def _t1():
                pltpu.make_async_copy(t1_hbm.at[srcb.at[b]],
                                      rows_[sl], gs_[sl]).wait()

        def g_sd(sl, b):
            return pltpu.make_async_copy(sdst_hbm.at[dstb.at[b]],
                                         sd_[sl], ds_[sl])

        def g_sc(sl, b):
            return pltpu.make_async_copy(rows_[sl], acc.at[dstb.at[b]],
                                         cs_[sl])

        def compute(sl):
            rows = rows_[sl]
            sd = sd_[sl]

            @plsc.parallel_loop(0, B, 1, unroll=4)
            def edge(e):
                sv = rows[e, pl.ds(128, 16)]
                dv = sd[e, :]
                z = sv + dv
                w = jnp.exp(jnp.maximum(z, 0.2 * z))
                rows[e, pl.ds(128, 16)] = w
                for hh in range(128 // (16 * cph)):
                    lane = g * lpg + (loff + hh)
                    wk = w.at[jnp.full((16,), lane, jnp.int32)].get(
                        mode="promise_in_bounds")
                    for p in range(cph):
                        kk = hh * cph + p
                        rows[e, pl.ds(16 * kk, 16)] = (
                            rows[e, pl.ds(16 * kk, 16)] * wk)

        def step(b, sl):
            other = 1 - sl
            nb = b + 1

            @pl.when(nb < hb)
            def _issue():
                @pl.when(nb >= 2)
                def _drain():
                    g_sc(other, 0).wait()
                g_rows_start(other, nb)
                g_sd(other, nb).start()

            g_rows_wait(sl, b)
            g_sd(sl, b).wait()
            compute(sl)
            g_sc(sl, b).start(add=True)

        def body(i, c):
            step(2 * i, 0)
            step(2 * i + 1, 1)
            return c

        base0 = s * RPT
        pltpu.sync_copy(src_hbm.at[pl.ds(s * nbatch, hb)], srcb)
        pltpu.sync_copy(dst_hbm.at[pl.ds(s * nbatch, hb)], dstb)
        g_rows_start(0, 0)
        g_sd(0, 0).start()
        pltpu.sync_copy(z_hbm, acc.at[pl.ds(base0, RPT)])
        plsc.subcore_barrier()

        for half in range(2):
            if half == 1:
                rbase = s * nbatch + hb
                pltpu.sync_copy(src_hbm.at[pl.ds(rbase, hb)], srcb)
                pltpu.sync_copy(dst_hbm.at[pl.ds(rbase, hb)], dstb)
                g_rows_start(0, 0)
                g_sd(0, 0).start()
            lax.fori_loop(0, hb // 2, body, 0)
            g_sc(0, 0).wait()
            g_sc(1, 0).wait()

        plsc.subcore_barrier()
        pltpu.sync_copy(acc.at[pl.ds(base0, RPT)],
                        out_hbm.at[pl.ds(g * N_ACC + base0, RPT)])

    return k(t0, t1, sdst, src2d, dst2d, zrows)


def _gcn_sc(table, src3d, dst3d, zrows, width):
    BG = 128
    ept = E_PAD // (NC * NS)
    nbatch = ept // BG
    nslot = 2
    mesh = plsc.VectorSubcoreMesh(core_axis_name="c", subcore_axis_name="s")

    @functools.partial(
        pl.kernel,
        out_type=jax.ShapeDtypeStruct((NC * N_ACC, width), jnp.float32),
        mesh=mesh,
        scratch_types=[
            pltpu.VMEM((nbatch, BG), jnp.int32),
            pltpu.VMEM((nbatch, BG), jnp.int32),
        ] + [pltpu.VMEM((BG, width), jnp.float32)] * nslot
          + [pltpu.VMEM_SHARED((N_ACC, width), jnp.float32)]
          + [pltpu.SemaphoreType.DMA] * (2 * nslot),
        compiler_params=pltpu.CompilerParams(use_tc_tiling_on_sc=False),
    )
    def k(table_hbm, src_hbm, dst_hbm, z_hbm, out_hbm, srcb, dstb, *scr):
        rows_ = scr[:nslot]
        acc = scr[nslot]
        gs_ = scr[nslot + 1:2 * nslot + 1]
        cs_ = scr[2 * nslot + 1:]
        g = lax.axis_index("c")
        s = lax.axis_index("s")
        wid = g * NS + s

        rb = wid * nbatch
        pltpu.sync_copy(src_hbm.at[pl.ds(rb, nbatch)], srcb)
        pltpu.sync_copy(dst_hbm.at[pl.ds(rb, nbatch)], dstb)

        def g_rows(sl, b):
            return pltpu.make_async_copy(table_hbm.at[srcb.at[b]],
                                         rows_[sl], gs_[sl])

        def g_sc(sl, b):
            return pltpu.make_async_copy(rows_[sl], acc.at[dstb.at[b]],
                                         cs_[sl])

        g_rows(0, 0).start()
        base0 = s * RPT
        pltpu.sync_copy(z_hbm, acc.at[pl.ds(base0, RPT)])
        plsc.subcore_barrier()

        def step(t, sl):
            other = 1 - sl
            nb = t + 1

            @pl.when(nb < nbatch)
            def _issue():
                @pl.when(nb >= 2)
                def _drain():
                    g_sc(other, 0).wait()
                g_rows(other, nb).start()

            g_rows(sl, t).wait()
            g_sc(sl, t).start(add=True)

        def body(i, c):
            step(2 * i, 0)
            step(2 * i + 1, 1)
            return c
        lax.fori_loop(0, nbatch // 2, body, 0)
        g_sc(0, 0).wait()
        g_sc(1, 0).wait()

        plsc.subcore_barrier()
        pltpu.sync_copy(acc.at[pl.ds(base0, RPT)],
                        out_hbm.at[pl.ds(g * N_ACC + base0, RPT)])

    return k(table, src3d, dst3d, zrows)


RB = 1000
F32 = jnp.float32


def _node_spec(width):
    return pl.BlockSpec((RB, width), lambda i: (i, 0))


def _full_spec(r, c):
    return pl.BlockSpec((r, c), lambda i: (0, 0))


def _out(width):
    return jax.ShapeDtypeStruct((N, width), F32)


def _tc1(x, wcat, acat):
    def body(x_ref, w_ref, a_ref, t0_ref, t1_ref, sd_ref, mm_ref):
        xb = x_ref[...]
        hcat = jnp.dot(xb, w_ref[...], preferred_element_type=F32)
        h1 = hcat[:, :256]
        mm_ref[...] = hcat[:, 256:]
        sall = jnp.dot(h1, a_ref[...], preferred_element_type=F32)
        zp = jnp.zeros((RB, 8), F32)
        t0_ref[...] = jnp.concatenate([h1[:, :128], sall[:, :8], zp], axis=1)
        t1_ref[...] = jnp.concatenate([h1[:, 128:], sall[:, :8], zp], axis=1)
        sd_ref[...] = jnp.concatenate([sall[:, 8:], zp], axis=1)

    return pl.pallas_call(
        body,
        grid=(N // RB,),
        in_specs=[_node_spec(1024), _full_spec(1024, 512), _full_spec(256, 16)],
        out_specs=[_node_spec(GATW), _node_spec(GATW), _node_spec(16),
                   _node_spec(256)],
        out_shape=[_out(GATW), _out(GATW), _out(16), _out(256)],
    )(x, wcat, acat)


def _tc2(a0, a1, t0, t1, sd1, mm1, pv, w2cat):
    def body(a0_ref, a1_ref, t0_ref, t1_ref, sd_ref, mm_ref, pv_ref, w_ref,
             t2_ref, h2_ref, mm2_ref, dv_ref):
        a0b, a1b = a0_ref[...], a1_ref[...]
        t0b, t1b = t0_ref[...], t1_ref[...]
        rep = (lax.broadcasted_iota(jnp.int32, (4, 128), 1) // 32 ==
               lax.broadcasted_iota(jnp.int32, (4, 128), 0)).astype(F32)
        ssrc = t0b[:, 128:136]
        sdst = sd_ref[...][:, :8]
        z = ssrc + sdst
        wself = jnp.exp(jnp.maximum(z, 0.2 * z))
        deg = a0b[:, 136:137] + 1.0
        dinv = lax.rsqrt(deg)
        num0 = a0b[:, :128] + t0b[:, :128] * jnp.dot(wself[:, :4], rep)
        den0 = jnp.dot(a0b[:, 128:132] + wself[:, :4], rep)
        num1 = a1b[:, :128] + t1b[:, :128] * jnp.dot(wself[:, 4:], rep)
        den1 = jnp.dot(a1b[:, 132:136] + wself[:, 4:], rep)
        gat = jnp.concatenate([num0 / den0, num1 / den1], axis=1)
        pvb = pv_ref[...]
        o = gat + mm_ref[...] + pvb[0] + pvb[1]
        mu = o.mean(-1, keepdims=True)
        var = ((o - mu) ** 2).mean(-1, keepdims=True)
        h = jnp.maximum((o - mu) * lax.rsqrt(var + 1e-5) * pvb[2] + pvb[3], 0.0)
        hcat = jnp.dot(h, w_ref[...], preferred_element_type=F32)
        h2 = hcat[:, :128]
        h2_ref[...] = h2
        mm2_ref[...] = hcat[:, 128:]
        t2_ref[...] = h2 * dinv
        dv_ref[...] = jnp.broadcast_to(dinv, (RB, 8))

    return pl.pallas_call(
        body,
        grid=(N // RB,),
        in_specs=[_node_spec(GATW), _node_spec(GATW), _node_spec(GATW),
                  _node_spec(GATW), _node_spec(16), _node_spec(256),
                  _full_spec(4, 256), _full_spec(256, 256)],
        out_specs=[_node_spec(128), _node_spec(128), _node_spec(128),
                   _node_spec(8)],
        out_shape=[_out(128), _out(128), _out(128), _out(8)],
    )(a0, a1, t0, t1, sd1, mm1, pv, w2cat)


def _tc3(p0, p1, h2, mm2, dv8, pv, w3, acat3, wm3):
    def body(p0_ref, p1_ref, h2_ref, mm2_ref, dv_ref, pv_ref, w3_ref, a_ref,
             wm_ref, t30_ref, t31_ref, t32_ref, t33_ref, sd_ref, mm3_ref):
        dinv = dv_ref[...][:, :1]
        gcn = (p0_ref[...] + p1_ref[...]) * dinv + h2_ref[...] * dinv * dinv
        pvb = pv_ref[...]
        o = gcn + mm2_ref[...] + pvb[0] + pvb[1]
        mu = o.mean(-1, keepdims=True)
        var = ((o - mu) ** 2).mean(-1, keepdims=True)
        h = jnp.maximum((o - mu) * lax.rsqrt(var + 1e-5) * pvb[2] + pvb[3], 0.0)
        h3 = jnp.dot(h, w3_ref[...], preferred_element_type=F32)
        sall = jnp.dot(h3, a_ref[...], preferred_element_type=F32)
        mm3_ref[...] = jnp.dot(h, wm_ref[...], preferred_element_type=F32)
        zp = jnp.zeros((RB, 8), F32)
        for g, tref in enumerate((t30_ref, t31_ref, t32_ref, t33_ref)):
            tref[...] = jnp.concatenate(
                [h3[:, 128 * g:128 * (g + 1)], sall[:, :8], zp], axis=1)
        sd_ref[...] = jnp.concatenate([sall[:, 8:], zp], axis=1)

    return pl.pallas_call(
        body,
        grid=(N // RB,),
        in_specs=[_node_spec(128), _node_spec(128), _node_spec(128),
                  _node_spec(128), _node_spec(8), _full_spec(4, 128),
                  _full_spec(128, 512), _full_spec(512, 16),
                  _full_spec(128, 64)],
        out_specs=[_node_spec(GATW)] * 4 + [_node_spec(16), _node_spec(64)],
        out_shape=[_out(GATW)] * 4 + [_out(16), _out(64)],
    )(p0, p1, h2, mm2, dv8, pv, w3, acat3, wm3)


def _tc4(accs, t3s, sd3, mm3, dv8, pv):
    def body(a0_ref, a1_ref, a2_ref, a3_ref, t0_ref, t1_ref, t2_ref, t3_ref,
             sd_ref, mm_ref, dv_ref, pv_ref, t4_ref, h4_ref):
        arefs = (a0_ref, a1_ref, a2_ref, a3_ref)
        trefs = (t0_ref, t1_ref, t2_ref, t3_ref)
        rep = (lax.broadcasted_iota(jnp.int32, (2, 128), 1) // 64 ==
               lax.broadcasted_iota(jnp.int32, (2, 128), 0)).astype(F32)
        mean8 = (lax.broadcasted_iota(jnp.int32, (512, 64), 0) % 64 ==
                 lax.broadcasted_iota(jnp.int32, (512, 64), 1)).astype(F32) / 8.0
        ssrc = t0_ref[...][:, 128:136]
        z = ssrc + sd_ref[...][:, :8]
        wself = jnp.exp(jnp.maximum(z, 0.2 * z))
        ratios = []
        for g in range(4):
            ab, tb = arefs[g][...], trefs[g][...]
            ws2 = wself[:, 2 * g:2 * g + 2]
            num = ab[:, :128] + tb[:, :128] * jnp.dot(ws2, rep)
            den = jnp.dot(ab[:, 128 + 2 * g:130 + 2 * g] + ws2, rep)
            ratios.append(num / den)
        rat = jnp.concatenate(ratios, axis=1)
        out64 = jnp.dot(rat, mean8, preferred_element_type=F32)
        pvb = pv_ref[...]
        o = out64 + mm_ref[...] + pvb[0] + pvb[1]
        mu = o.mean(-1, keepdims=True)
        var = ((o - mu) ** 2).mean(-1, keepdims=True)
        h = jnp.maximum((o - mu) * lax.rsqrt(var + 1e-5) * pvb[2] + pvb[3], 0.0)
        t4_ref[...] = h * dv_ref[...][:, :1]
        h4_ref[...] = h

    return pl.pallas_call(
        body,
        grid=(N // RB,),
        in_specs=[_node_spec(GATW)] * 8 + [_node_spec(16), _node_spec(64),
                                           _node_spec(8), _full_spec(4, 64)],
        out_specs=[_node_spec(64), _node_spec(64)],
        out_shape=[_out(64), _out(64)],
    )(*accs, *t3s, sd3, mm3, dv8, pv)


def _tc5(p0, p1, h4, dv8, w4, wm4, bsum):
    def body(p0_ref, p1_ref, h4_ref, dv_ref, w4_ref, wm_ref, b_ref, o_ref):
        dinv = dv_ref[...][:, :1]
        h4b = h4_ref[...]
        gcn = (p0_ref[...] + p1_ref[...]) * dinv + h4b * dinv * dinv
        o_ref[...] = (jnp.dot(gcn, w4_ref[...], preferred_element_type=F32) +
                      jnp.dot(h4b, wm_ref[...], preferred_element_type=F32) +
                      b_ref[...])

    return pl.pallas_call(
        body,
        grid=(N // RB,),
        in_specs=[_node_spec(64), _node_spec(64), _node_spec(64),
                  _node_spec(8), _full_spec(64, 2), _full_spec(64, 2),
                  _full_spec(1, 2)],
        out_specs=_node_spec(2),
        out_shape=_out(2),
    )(p0, p1, h4, dv8, w4, wm4, bsum)


def _acat(a_src, a_dst, out_ch):
    c = 8 * out_ch
    hot = (jnp.arange(c)[:, None] // out_ch == jnp.arange(8)[None, :]
           ).astype(F32)
    return jnp.concatenate([a_src.reshape(-1)[:, None] * hot,
                            a_dst.reshape(-1)[:, None] * hot], axis=1)


def kernel(x, edge_index, W1, a_src1, a_dst1, b1, Wm1, bm1, g0, be0,
           W2, b2, Wm2, bm2, g1, be1, W3, a_src3, a_dst3, b3, Wm3, bm3,
           g2, be2, W4, b4, Wm4, bm4):
    npad = E_PAD - E
    srcp = jnp.concatenate([edge_index[0].astype(jnp.int32),
                            jnp.zeros((npad,), jnp.int32)])
    dstp = jnp.concatenate([edge_index[1].astype(jnp.int32),
                            jnp.full((npad,), N, jnp.int32)])
    src, dst = srcp.reshape(-1, B), dstp.reshape(-1, B)
    srcg, dstg = srcp.reshape(-1, 128), dstp.reshape(-1, 128)
    sdpad = jnp.zeros((N_ACC - N, 16), F32)
    zgat = jnp.zeros((RPT, GATW), F32)
    z128 = jnp.zeros((RPT, 128), F32)
    z64 = jnp.zeros((RPT, 64), F32)

    t0, t1, sd1, mm1 = _tc1(x, jnp.concatenate([W1, Wm1], axis=1),
                            _acat(a_src1, a_dst1, 32))
    sd1e = jnp.concatenate([sd1, sdpad], axis=0)
    acc1 = _gat_sc(t0, t1, sd1e, src, dst, zgat, lpg=4, loff=0, cph=2)
    t2, h2, mm2, dv8 = _tc2(acc1[:N], acc1[N_ACC:N_ACC + N], t0, t1, sd1, mm1,
                            jnp.stack([b1, bm1, g0, be0]),
                            jnp.concatenate([W2, Wm2], axis=1))

    p2 = _gcn_sc(t2, srcg, dstg, z128, 128)
    t30, t31, t32, t33, sd3, mm3 = _tc3(p2[:N], p2[N_ACC:N_ACC + N], h2,
                                        mm2, dv8,
                                        jnp.stack([b2, bm2, g1, be1]),
                                        W3, _acat(a_src3, a_dst3, 64), Wm3)

    sd3e = jnp.concatenate([sd3, sdpad], axis=0)
    accA = _gat_sc(t30, t31, sd3e, src, dst, zgat, lpg=2, loff=0, cph=4)
    accB = _gat_sc(t32, t33, sd3e, src, dst, zgat, lpg=2, loff=4, cph=4)
    t4, h4 = _tc4((accA[:N], accA[N_ACC:N_ACC + N],
                   accB[:N], accB[N_ACC:N_ACC + N]),
                  (t30, t31, t32, t33), sd3, mm3, dv8,
                  jnp.stack([b3, bm3, g2, be2]))

    p4 = _gcn_sc(t4, srcg, dstg, z64, 64)
    return _tc5(p4[:N], p4[N_ACC:N_ACC + N], h4, dv8, W4, Wm4,
                (b4 + bm4)[None])
